# Initial kernel scaffold; baseline (speedup 1.0000x reference)
import jax
import jax.numpy as jnp
from jax.experimental import pallas as pl


def kernel(x, edge_index, edge_attr, params):
    raise NotImplementedError("write your pallas kernel here")


# speedup vs baseline: 1.3635x; 1.3635x over previous
import functools

import jax
import jax.numpy as jnp
from jax import lax
from jax.experimental import pallas as pl
from jax.experimental.pallas import tpu as pltpu
from jax.experimental.pallas import tpu_sc as plsc

_N = 10000
_E = 320000
_H = 32
_EPS = 1e-5

_NC = 2
_NS = 16
_NW = _NC * _NS
_EPW = _E // _NW
_CH = 80
_NCH = _EPW // _CH
_RPT = _N // _NS

_BR = 4000
_F32 = jnp.float32


@functools.cache
def _sc_mesh():
    return plsc.VectorSubcoreMesh(core_axis_name="c", subcore_axis_name="s",
                                  num_cores=_NC, num_subcores=_NS)


@functools.cache
def _sc_gather_combine():
    @functools.partial(
        pl.kernel,
        out_type=jax.ShapeDtypeStruct((_E, _H), _F32),
        mesh=_sc_mesh(),
        compiler_params=pltpu.CompilerParams(use_tc_tiling_on_sc=False),
        scratch_types=[
            pltpu.VMEM((_CH,), jnp.int32),
            pltpu.VMEM((_CH,), jnp.int32),
            pltpu.VMEM((_CH, _H), _F32),
            pltpu.VMEM((_CH, _H), _F32),
            pltpu.SemaphoreType.DMA,
        ],
    )
    def gather_combine(a_hbm, b_hbm, dst_hbm, src_hbm, out_hbm,
                       ia, ib, ra, rb, sem):
        wid = lax.axis_index("s") * _NC + lax.axis_index("c")
        wstart = wid * _EPW

        def chunk(i, carry):
            base = wstart + i * _CH
            pltpu.sync_copy(dst_hbm.at[pl.ds(base, _CH)], ia)
            pltpu.sync_copy(src_hbm.at[pl.ds(base, _CH)], ib)
            cp1 = pltpu.async_copy(a_hbm.at[ia], ra, sem)
            cp2 = pltpu.async_copy(b_hbm.at[ib], rb, sem)
            cp1.wait()
            cp2.wait()

            def addrow(r, c2):
                for h in range(2):
                    sl = pl.ds(h * 16, 16)
                    ra[r, sl] = ra[r, sl] + rb[r, sl]
                return c2

            lax.fori_loop(0, _CH, addrow, 0)
            pltpu.sync_copy(ra, out_hbm.at[pl.ds(base, _CH)])
            return carry

        lax.fori_loop(0, _NCH, chunk, 0)

    return gather_combine


@functools.cache
def _sc_scatter_msg():
    @functools.partial(
        pl.kernel,
        out_type=jax.ShapeDtypeStruct((_NC, _N, _H), _F32),
        mesh=_sc_mesh(),
        compiler_params=pltpu.CompilerParams(use_tc_tiling_on_sc=False),
        scratch_types=[
            pltpu.VMEM((_CH,), jnp.int32),
            pltpu.VMEM((_CH, _H), _F32),
            pltpu.VMEM((_RPT, _H), _F32),
            pltpu.VMEM((3, _H), _F32),
            pltpu.VMEM_SHARED((_N, _H), _F32),
        ],
    )
    def scatter_msg(z_hbm, dst_hbm, ac_hbm, out_hbm,
                    idxv, zrows, tbuf, acv, acc):
        cid = lax.axis_index("c")
        sid = lax.axis_index("s")
        wid = sid * _NC + cid

        def zrow(r, carry):
            for h in range(2):
                tbuf[r, pl.ds(h * 16, 16)] = jnp.zeros((16,), _F32)
            return carry

        lax.fori_loop(0, _RPT, zrow, 0)
        pltpu.sync_copy(tbuf, acc.at[pl.ds(sid * _RPT, _RPT)])
        plsc.subcore_barrier()

        pltpu.sync_copy(ac_hbm, acv)
        a_lo = acv[0, pl.ds(0, 16)]
        a_hi = acv[0, pl.ds(16, 16)]
        c_lo = acv[1, pl.ds(0, 16)]
        c_hi = acv[1, pl.ds(16, 16)]
        p_lo = acv[2, pl.ds(0, 16)]
        p_hi = acv[2, pl.ds(16, 16)]
        wstart = wid * _EPW

        def chunk(i, carry):
            base = wstart + i * _CH
            pltpu.sync_copy(dst_hbm.at[pl.ds(base, _CH)], idxv)
            pltpu.sync_copy(z_hbm.at[pl.ds(base, _CH)], zrows)

            def prow(r, c2):
                for h, (av, cv, pv) in enumerate(((a_lo, c_lo, p_lo),
                                                  (a_hi, c_hi, p_hi))):
                    sl = pl.ds(h * 16, 16)
                    v = zrows[r, sl] * av + cv
                    zrows[r, sl] = jnp.where(v >= 0.0, v, pv * v)
                return c2

            lax.fori_loop(0, _CH, prow, 0)
            pltpu.sync_copy(zrows, acc.at[idxv], add=True)
            return carry

        lax.fori_loop(0, _NCH, chunk, 0)
        plsc.subcore_barrier()

        pltpu.sync_copy(acc.at[pl.ds(sid * _RPT, _RPT)], tbuf)
        pltpu.sync_copy(tbuf, out_hbm.at[cid, pl.ds(sid * _RPT, _RPT)])

    return scatter_msg


@functools.cache
def _sc_count_dst():
    @functools.partial(
        pl.kernel,
        out_type=jax.ShapeDtypeStruct((_NC, _N, 16), _F32),
        mesh=_sc_mesh(),
        compiler_params=pltpu.CompilerParams(use_tc_tiling_on_sc=False),
        scratch_types=[
            pltpu.VMEM((_CH,), jnp.int32),
            pltpu.VMEM((_CH, 16), _F32),
            pltpu.VMEM((_RPT, 16), _F32),
            pltpu.VMEM_SHARED((_N, 16), _F32),
        ],
    )
    def count_dst(dst_hbm, out_hbm, idxv, ones, tbuf, acc):
        cid = lax.axis_index("c")
        sid = lax.axis_index("s")
        wid = sid * _NC + cid

        def zrow(r, carry):
            tbuf[r, pl.ds(0, 16)] = jnp.zeros((16,), _F32)
            return carry

        lax.fori_loop(0, _RPT, zrow, 0)
        pltpu.sync_copy(tbuf, acc.at[pl.ds(sid * _RPT, _RPT)])
        plsc.subcore_barrier()

        lane = lax.broadcasted_iota(jnp.int32, (16,), 0)
        one_row = jnp.where(lane == 0, 1.0, 0.0).astype(_F32)

        def orow(r, carry):
            ones[r, pl.ds(0, 16)] = one_row
            return carry

        lax.fori_loop(0, _CH, orow, 0)
        wstart = wid * _EPW

        def chunk(i, carry):
            base = wstart + i * _CH
            pltpu.sync_copy(dst_hbm.at[pl.ds(base, _CH)], idxv)
            pltpu.sync_copy(ones, acc.at[idxv], add=True)
            return carry

        lax.fori_loop(0, _NCH, chunk, 0)
        plsc.subcore_barrier()

        pltpu.sync_copy(acc.at[pl.ds(sid * _RPT, _RPT)], tbuf)
        pltpu.sync_copy(tbuf, out_hbm.at[cid, pl.ds(sid * _RPT, _RPT)])

    return count_dst


def _gather_combine(a, b, dst, src):
    return _sc_gather_combine()(a, b, dst, src)


def _scatter_msg(z, dst, ac):
    return _sc_scatter_msg()(z, dst, ac)


def _count_dst(dst):
    return _sc_count_dst()(dst)


def _lin_stats(e, w, b, g=None):
    n, de = e.shape
    h = w.shape[1]
    nb = n // _BR

    def body(*refs):
        if g is None:
            e_ref, w_ref, b_ref, z_ref, s_ref = refs
            z = jnp.dot(e_ref[...], w_ref[...],
                        preferred_element_type=_F32) + b_ref[...]
        else:
            g_ref, e_ref, w_ref, b_ref, z_ref, s_ref = refs
            z = g_ref[...] + jnp.dot(e_ref[...], w_ref[...],
                                     preferred_element_type=_F32) + b_ref[...]
        z_ref[...] = z

        @pl.when(pl.program_id(0) == 0)
        def _init():
            s_ref[...] = jnp.zeros_like(s_ref)

        s_ref[...] += jnp.stack([jnp.sum(z, 0), jnp.sum(z * z, 0)])

    in_specs = [
        pl.BlockSpec((_BR, de), lambda i: (i, 0)),
        pl.BlockSpec((de, h), lambda i: (0, 0)),
        pl.BlockSpec((1, h), lambda i: (0, 0)),
    ]
    args = [e, w, b]
    if g is not None:
        in_specs.insert(0, pl.BlockSpec((_BR, h), lambda i: (i, 0)))
        args.insert(0, g)
    return pl.pallas_call(
        body,
        grid=(nb,),
        in_specs=in_specs,
        out_specs=[pl.BlockSpec((_BR, h), lambda i: (i, 0)),
                   pl.BlockSpec((2, h), lambda i: (0, 0))],
        out_shape=[jax.ShapeDtypeStruct((n, h), _F32),
                   jax.ShapeDtypeStruct((2, h), _F32)],
    )(*args)


def _act_lin(z1, vec, w2, with_stats):
    n, h = z1.shape
    dout = w2.shape[1]
    nb = n // _BR

    def body(z_ref, v_ref, w_ref, *orefs):
        a = v_ref[0:1, :]
        c = v_ref[1:2, :]
        alpha = v_ref[2:3, :]
        b2 = v_ref[3:4, :dout]
        t = z_ref[...] * a + c
        t = jnp.where(t >= 0.0, t, alpha * t)
        z2 = jnp.dot(t, w_ref[...], preferred_element_type=_F32) + b2
        orefs[0][...] = z2
        if with_stats:
            @pl.when(pl.program_id(0) == 0)
            def _init():
                orefs[1][...] = jnp.zeros_like(orefs[1])
            orefs[1][...] += jnp.stack([jnp.sum(z2, 0), jnp.sum(z2 * z2, 0)])

    out_specs = [pl.BlockSpec((_BR, dout), lambda i: (i, 0))]
    out_shape = [jax.ShapeDtypeStruct((n, dout), _F32)]
    if with_stats:
        out_specs.append(pl.BlockSpec((2, dout), lambda i: (0, 0)))
        out_shape.append(jax.ShapeDtypeStruct((2, dout), _F32))
    res = pl.pallas_call(
        body,
        grid=(nb,),
        in_specs=[pl.BlockSpec((_BR, h), lambda i: (i, 0)),
                  pl.BlockSpec((4, h), lambda i: (0, 0)),
                  pl.BlockSpec((h, dout), lambda i: (0, 0))],
        out_specs=out_specs,
        out_shape=out_shape,
    )(z1, vec, w2)
    return res if with_stats else res[0]


def _act_residual(z, vec, base):
    n, h = z.shape
    nb = n // _BR

    def body(*refs):
        if base is None:
            z_ref, v_ref, o_ref = refs
        else:
            z_ref, v_ref, b_ref, o_ref = refs
        t = z_ref[...] * v_ref[0:1, :] + v_ref[1:2, :]
        t = jnp.where(t >= 0.0, t, v_ref[2:3, :] * t)
        if base is not None:
            t = b_ref[...] + t
        o_ref[...] = t

    in_specs = [pl.BlockSpec((_BR, h), lambda i: (i, 0)),
                pl.BlockSpec((3, h), lambda i: (0, 0))]
    args = [z, vec]
    if base is not None:
        in_specs.append(pl.BlockSpec((_BR, h), lambda i: (i, 0)))
        args.append(base)
    return pl.pallas_call(
        body,
        grid=(nb,),
        in_specs=in_specs,
        out_specs=pl.BlockSpec((_BR, h), lambda i: (i, 0)),
        out_shape=jax.ShapeDtypeStruct((n, h), _F32),
    )(*args)


def _mlp2_block(x, w1, vec, w2):
    n, din = x.shape
    h = w1.shape[1]

    def bn_act(z, gm, bt, al):
        m = jnp.mean(z, 0, keepdims=True)
        v = jnp.mean(z * z, 0, keepdims=True) - m * m
        t = (z - m) * lax.rsqrt(v + _EPS) * gm + bt
        return jnp.where(t >= 0.0, t, al * t)

    def body(x_ref, w1_ref, v_ref, w2_ref, o_ref):
        z1 = jnp.dot(x_ref[...], w1_ref[...],
                     preferred_element_type=_F32) + v_ref[0:1, :]
        t = bn_act(z1, v_ref[1:2, :], v_ref[2:3, :], v_ref[3:4, :])
        z2 = jnp.dot(t, w2_ref[...],
                     preferred_element_type=_F32) + v_ref[4:5, :]
        o_ref[...] = bn_act(z2, v_ref[5:6, :], v_ref[6:7, :], v_ref[7:8, :])

    return pl.pallas_call(
        body,
        in_specs=[pl.BlockSpec((n, din), lambda: (0, 0)),
                  pl.BlockSpec((din, h), lambda: (0, 0)),
                  pl.BlockSpec((8, h), lambda: (0, 0)),
                  pl.BlockSpec((h, h), lambda: (0, 0))],
        out_specs=pl.BlockSpec((n, h), lambda: (0, 0)),
        out_shape=jax.ShapeDtypeStruct((n, h), _F32),
    )(x, w1, vec, w2)


def _node_update(node, accp, cntp, w1a, w1b, vec, w2, wproj, signs):
    n, h = node.shape
    pw = wproj.shape[1]

    def bn_act(z, gm, bt, al):
        m = jnp.mean(z, 0, keepdims=True)
        v = jnp.mean(z * z, 0, keepdims=True) - m * m
        t = (z - m) * lax.rsqrt(v + _EPS) * gm + bt
        return jnp.where(t >= 0.0, t, al * t)

    def body(nd_ref, ac_ref, ct_ref, wa_ref, wb_ref, v_ref, w2_ref,
             wp_ref, sg_ref, on_ref, op_ref):
        cnt = ct_ref[0] + ct_ref[1]
        inv = 1.0 / jnp.maximum(cnt[:, 0:1], 1.0)
        agg = (ac_ref[0] + ac_ref[1]) * inv
        nd = nd_ref[...]
        z1 = (jnp.dot(nd, wa_ref[...], preferred_element_type=_F32)
              + jnp.dot(agg, wb_ref[...], preferred_element_type=_F32)
              + v_ref[0:1, :])
        t = bn_act(z1, v_ref[1:2, :], v_ref[2:3, :], v_ref[3:4, :])
        z2 = jnp.dot(t, w2_ref[...],
                     preferred_element_type=_F32) + v_ref[4:5, :]
        nd_new = nd + bn_act(z2, v_ref[5:6, :], v_ref[6:7, :], v_ref[7:8, :])
        on_ref[...] = nd_new
        op_ref[...] = jnp.dot(nd_new, wp_ref[...],
                              preferred_element_type=_F32) * sg_ref[...]

    return pl.pallas_call(
        body,
        in_specs=[pl.BlockSpec((n, h), lambda: (0, 0)),
                  pl.BlockSpec((2, n, h), lambda: (0, 0, 0)),
                  pl.BlockSpec((2, n, 16), lambda: (0, 0, 0)),
                  pl.BlockSpec((h, h), lambda: (0, 0)),
                  pl.BlockSpec((h, h), lambda: (0, 0)),
                  pl.BlockSpec((8, h), lambda: (0, 0)),
                  pl.BlockSpec((h, h), lambda: (0, 0)),
                  pl.BlockSpec((h, pw), lambda: (0, 0)),
                  pl.BlockSpec((1, pw), lambda: (0, 0))],
        out_specs=[pl.BlockSpec((n, h), lambda: (0, 0)),
                   pl.BlockSpec((n, pw), lambda: (0, 0))],
        out_shape=[jax.ShapeDtypeStruct((n, h), _F32),
                   jax.ShapeDtypeStruct((n, pw), _F32)],
    )(node, accp, cntp, w1a, w1b, vec, w2, wproj, signs)


def _proj_call(node, wproj):
    n, h = node.shape
    pw = wproj.shape[1]

    def body(nd_ref, wp_ref, o_ref):
        o_ref[...] = jnp.dot(nd_ref[...], wp_ref[...],
                             preferred_element_type=_F32)

    return pl.pallas_call(
        body,
        in_specs=[pl.BlockSpec((n, h), lambda: (0, 0)),
                  pl.BlockSpec((h, pw), lambda: (0, 0))],
        out_specs=pl.BlockSpec((n, pw), lambda: (0, 0)),
        out_shape=jax.ShapeDtypeStruct((n, pw), _F32),
    )(node, wproj)


def _fold_bn(stats, n, bn, alpha, b2=None):
    m = stats[0] / n
    v = stats[1] / n - m * m
    a = bn["gamma"] * lax.rsqrt(v + _EPS)
    c = bn["beta"] - m * a
    h = a.shape[0]
    al = jnp.full((h,), alpha, _F32)
    if b2 is None:
        return jnp.stack([a, c, al])
    b2p = jnp.zeros((h,), _F32).at[: b2.shape[0]].set(b2)
    return jnp.stack([a, c, al, b2p])


def _mlp_vec(p):
    return jnp.stack([
        p["lin1"]["b"], p["bn1"]["gamma"], p["bn1"]["beta"],
        jnp.full((_H,), p["pr1"]["alpha"], _F32),
        p["lin2"]["b"], p["bn2"]["gamma"], p["bn2"]["beta"],
        jnp.full((_H,), p["pr2"]["alpha"], _F32),
    ])


def kernel(x, edge_index, edge_attr, params):
    src = edge_index[0]
    dst = edge_index[1]
    layers = params["layers"]
    fin = params["final"]

    pe = params["node_emb"]
    node = _mlp2_block(x, pe["lin1"]["w"], _mlp_vec(pe), pe["lin2"]["w"])

    wm1 = layers[0]["msg"]["lin1"]["w"]
    proj = _proj_call(node, jnp.concatenate([wm1[:_H], wm1[_H:2 * _H]], 1))

    pg = params["edge_emb"]
    z1, s1 = _lin_stats(edge_attr, pg["lin1"]["w"], pg["lin1"]["b"][None])
    v1 = _fold_bn(s1, _E, pg["bn1"], pg["pr1"]["alpha"], pg["lin2"]["b"])
    z2, s2 = _act_lin(z1, v1, pg["lin2"]["w"], True)
    edge = _act_residual(z2, _fold_bn(s2, _E, pg["bn2"], pg["pr2"]["alpha"]),
                         None)

    cntp = _count_dst(dst)

    for li, lp in enumerate(layers):
        mp, np_, ep = lp["msg"], lp["node"], lp["edge"]

        g1 = _gather_combine(proj[:, :_H], proj[:, _H:2 * _H], dst, src)
        wm = mp["lin1"]["w"]
        z1, s1 = _lin_stats(edge, wm[2 * _H:], mp["lin1"]["b"][None], g=g1)
        v1 = _fold_bn(s1, _E, mp["bn1"], mp["pr1"]["alpha"], mp["lin2"]["b"])
        z2, s2 = _act_lin(z1, v1, mp["lin2"]["w"], True)
        ac = _fold_bn(s2, _E, mp["bn2"], mp["pr2"]["alpha"])
        accp = _scatter_msg(z2, dst, ac)

        we = ep["lin1"]["w"][_H:]
        if li + 1 < len(layers):
            wn = layers[li + 1]["msg"]["lin1"]["w"]
            wp = jnp.concatenate([we, we, wn[:_H], wn[_H:2 * _H]], 1)
            signs = jnp.concatenate([
                jnp.full((1, _H), -1.0, _F32), jnp.full((1, _H), 1.0, _F32),
                jnp.full((1, 2 * _H), 1.0, _F32)], 1)
        else:
            wf = fin["lin1"]["w"][:_H]
            wp = jnp.concatenate([we, we, wf, wf], 1)
            signs = jnp.concatenate([
                jnp.full((1, _H), -1.0, _F32), jnp.full((1, _H), 1.0, _F32),
                jnp.full((1, _H), 1.0, _F32), jnp.full((1, _H), -1.0, _F32)],
                1)
        wn1 = np_["lin1"]["w"]
        node, proj4 = _node_update(node, accp, cntp, wn1[:_H], wn1[_H:],
                                   _mlp_vec(np_), np_["lin2"]["w"], wp, signs)

        d = _gather_combine(proj4[:, :_H], proj4[:, _H:2 * _H], dst, src)
        z1, s1 = _lin_stats(edge, ep["lin1"]["w"][:_H], ep["lin1"]["b"][None],
                            g=d)
        v1 = _fold_bn(s1, _E, ep["bn1"], ep["pr1"]["alpha"], ep["lin2"]["b"])
        z2, s2 = _act_lin(z1, v1, ep["lin2"]["w"], True)
        edge = _act_residual(
            z2, _fold_bn(s2, _E, ep["bn2"], ep["pr2"]["alpha"]), edge)
        proj = proj4[:, 2 * _H:]

    gf = _gather_combine(proj[:, :_H], proj[:, _H:], dst, src)
    z1, s1 = _lin_stats(edge, fin["lin1"]["w"][_H:], fin["lin1"]["b"][None],
                        g=gf)
    v1 = _fold_bn(s1, _E, fin["bn1"], fin["pr1"]["alpha"], fin["lin2"]["b"])
    out = _act_lin(z1, v1, fin["lin2"]["w"], False)
    return out


# speedup vs baseline: 1.7575x; 1.2889x over previous
import functools

import jax
import jax.numpy as jnp
from jax import lax
from jax.experimental import pallas as pl
from jax.experimental.pallas import tpu as pltpu
from jax.experimental.pallas import tpu_sc as plsc

_N = 10000
_E = 320000
_H = 32
_EPS = 1e-5

_NC = 2
_NS = 16
_NW = _NC * _NS
_EPW = _E // _NW
_CH = 80
_NCH = _EPW // _CH
_RPT = _N // _NS

_BR = 4000
_F32 = jnp.float32


@functools.cache
def _sc_mesh():
    return plsc.VectorSubcoreMesh(core_axis_name="c", subcore_axis_name="s",
                                  num_cores=_NC, num_subcores=_NS)


@functools.cache
def _sc_gather_combine():
    @functools.partial(
        pl.kernel,
        out_type=jax.ShapeDtypeStruct((_E, _H), _F32),
        mesh=_sc_mesh(),
        compiler_params=pltpu.CompilerParams(use_tc_tiling_on_sc=False),
        scratch_types=[
            pltpu.VMEM((_NCH, _CH), jnp.int32),
            pltpu.VMEM((_NCH, _CH), jnp.int32),
            pltpu.VMEM((_CH, _H), _F32),
            pltpu.VMEM((_CH, _H), _F32),
            pltpu.VMEM((_CH, _H), _F32),
            pltpu.VMEM((_CH, _H), _F32),
            pltpu.SemaphoreType.DMA,
            pltpu.SemaphoreType.DMA,
        ],
    )
    def gather_combine(a_hbm, b_hbm, dst2_hbm, src2_hbm, out_hbm,
                       idx_a, idx_b, ra0, rb0, ra1, rb1, sem0, sem1):
        wid = lax.axis_index("s") * _NC + lax.axis_index("c")
        wrow = wid * _NCH
        wstart = wid * _EPW
        pltpu.sync_copy(dst2_hbm.at[pl.ds(wrow, _NCH)], idx_a)
        pltpu.sync_copy(src2_hbm.at[pl.ds(wrow, _NCH)], idx_b)

        def fire(c, ra_, rb_, sem_):
            pltpu.async_copy(a_hbm.at[idx_a.at[c]], ra_, sem_)
            pltpu.async_copy(b_hbm.at[idx_b.at[c]], rb_, sem_)

        def drain_process(c, ra_, rb_, sem_):
            pltpu.make_async_copy(a_hbm.at[idx_a.at[c]], ra_, sem_).wait()
            pltpu.make_async_copy(b_hbm.at[idx_b.at[c]], rb_, sem_).wait()

            def addrow(r, c2):
                for h in range(2):
                    sl = pl.ds(h * 16, 16)
                    ra_[r, sl] = ra_[r, sl] + rb_[r, sl]
                return c2

            lax.fori_loop(0, _CH, addrow, 0)
            pltpu.sync_copy(ra_, out_hbm.at[pl.ds(wstart + c * _CH, _CH)])

        fire(0, ra0, rb0, sem0)

        def pair(p, carry):
            c0 = 2 * p
            fire(c0 + 1, ra1, rb1, sem1)
            drain_process(c0, ra0, rb0, sem0)

            @pl.when(c0 + 2 < _NCH)
            def _():
                fire(c0 + 2, ra0, rb0, sem0)

            drain_process(c0 + 1, ra1, rb1, sem1)
            return carry

        lax.fori_loop(0, _NCH // 2, pair, 0)
        drain_process(_NCH - 1, ra0, rb0, sem0)

    return gather_combine


@functools.cache
def _sc_scatter_msg():
    @functools.partial(
        pl.kernel,
        out_type=jax.ShapeDtypeStruct((_NC, _N, _H), _F32),
        mesh=_sc_mesh(),
        compiler_params=pltpu.CompilerParams(use_tc_tiling_on_sc=False),
        scratch_types=[
            pltpu.VMEM((_NCH, _CH), jnp.int32),
            pltpu.VMEM((_CH, _H), _F32),
            pltpu.VMEM((_CH, _H), _F32),
            pltpu.VMEM((_RPT, _H), _F32),
            pltpu.VMEM((3, _H), _F32),
            pltpu.VMEM_SHARED((_N, _H), _F32),
            pltpu.SemaphoreType.DMA,
            pltpu.SemaphoreType.DMA,
            pltpu.SemaphoreType.DMA,
            pltpu.SemaphoreType.DMA,
        ],
    )
    def scatter_msg(z_hbm, dst2_hbm, ac_hbm, out_hbm,
                    idx_all, z0, z1, tbuf, acv, acc,
                    semz0, semz1, sems0, sems1):
        cid = lax.axis_index("c")
        sid = lax.axis_index("s")
        wid = sid * _NC + cid
        wrow = wid * _NCH
        wstart = wid * _EPW

        def zrow(r, carry):
            for h in range(2):
                tbuf[r, pl.ds(h * 16, 16)] = jnp.zeros((16,), _F32)
            return carry

        lax.fori_loop(0, _RPT, zrow, 0)
        pltpu.sync_copy(tbuf, acc.at[pl.ds(sid * _RPT, _RPT)])
        pltpu.sync_copy(dst2_hbm.at[pl.ds(wrow, _NCH)], idx_all)
        plsc.subcore_barrier()

        pltpu.sync_copy(ac_hbm, acv)
        a_lo = acv[0, pl.ds(0, 16)]
        a_hi = acv[0, pl.ds(16, 16)]
        c_lo = acv[1, pl.ds(0, 16)]
        c_hi = acv[1, pl.ds(16, 16)]
        p_lo = acv[2, pl.ds(0, 16)]
        p_hi = acv[2, pl.ds(16, 16)]

        def fire_z(c, z_, semz_):
            pltpu.async_copy(z_hbm.at[pl.ds(wstart + c * _CH, _CH)], z_,
                             semz_)

        def process(c, z_, semz_, sems_):
            pltpu.make_async_copy(
                z_hbm.at[pl.ds(wstart + c * _CH, _CH)], z_, semz_).wait()

            def prow(r, c2):
                for h, (av, cv, pv) in enumerate(((a_lo, c_lo, p_lo),
                                                  (a_hi, c_hi, p_hi))):
                    sl = pl.ds(h * 16, 16)
                    v = z_[r, sl] * av + cv
                    z_[r, sl] = jnp.where(v >= 0.0, v, pv * v)
                return c2

            lax.fori_loop(0, _CH, prow, 0)
            pltpu.async_copy(z_, acc.at[idx_all.at[c]], sems_, add=True)

        def wait_scat(c, z_, sems_):
            pltpu.make_async_copy(z_, acc.at[idx_all.at[c]], sems_).wait()

        fire_z(0, z0, semz0)
        fire_z(1, z1, semz1)

        def pair(p, carry):
            c0 = 2 * p
            process(c0, z0, semz0, sems0)

            @pl.when(c0 + 2 < _NCH)
            def _():
                wait_scat(c0, z0, sems0)
                fire_z(c0 + 2, z0, semz0)

            process(c0 + 1, z1, semz1, sems1)

            @pl.when(c0 + 3 < _NCH)
            def _():
                wait_scat(c0 + 1, z1, sems1)
                fire_z(c0 + 3, z1, semz1)

            return carry

        lax.fori_loop(0, _NCH // 2, pair, 0)
        process(_NCH - 1, z0, semz0, sems0)
        wait_scat(_NCH - 1, z0, sems0)
        wait_scat(_NCH - 2, z1, sems1)
        plsc.subcore_barrier()

        pltpu.sync_copy(acc.at[pl.ds(sid * _RPT, _RPT)], tbuf)
        pltpu.sync_copy(tbuf, out_hbm.at[cid, pl.ds(sid * _RPT, _RPT)])

    return scatter_msg


@functools.cache
def _sc_count_dst():
    @functools.partial(
        pl.kernel,
        out_type=jax.ShapeDtypeStruct((_NC, _N, 16), _F32),
        mesh=_sc_mesh(),
        compiler_params=pltpu.CompilerParams(use_tc_tiling_on_sc=False),
        scratch_types=[
            pltpu.VMEM((_NCH, _CH), jnp.int32),
            pltpu.VMEM((_CH, 16), _F32),
            pltpu.VMEM((_RPT, 16), _F32),
            pltpu.VMEM_SHARED((_N, 16), _F32),
            pltpu.SemaphoreType.DMA,
        ],
    )
    def count_dst(dst2_hbm, out_hbm, idx_all, ones, tbuf, acc, sem):
        cid = lax.axis_index("c")
        sid = lax.axis_index("s")
        wid = sid * _NC + cid
        wrow = wid * _NCH

        def zrow(r, carry):
            tbuf[r, pl.ds(0, 16)] = jnp.zeros((16,), _F32)
            return carry

        lax.fori_loop(0, _RPT, zrow, 0)
        pltpu.sync_copy(tbuf, acc.at[pl.ds(sid * _RPT, _RPT)])
        pltpu.sync_copy(dst2_hbm.at[pl.ds(wrow, _NCH)], idx_all)

        lane = lax.broadcasted_iota(jnp.int32, (16,), 0)
        one_row = jnp.where(lane == 0, 1.0, 0.0).astype(_F32)

        def orow(r, carry):
            ones[r, pl.ds(0, 16)] = one_row
            return carry

        lax.fori_loop(0, _CH, orow, 0)
        plsc.subcore_barrier()

        def chunk(c, carry):
            pltpu.async_copy(ones, acc.at[idx_all.at[c]], sem, add=True)
            return carry

        lax.fori_loop(0, _NCH, chunk, 0)

        def drain(c, carry):
            pltpu.make_async_copy(ones, acc.at[idx_all.at[c]], sem).wait()
            return carry

        lax.fori_loop(0, _NCH, drain, 0)
        plsc.subcore_barrier()

        pltpu.sync_copy(acc.at[pl.ds(sid * _RPT, _RPT)], tbuf)
        pltpu.sync_copy(tbuf, out_hbm.at[cid, pl.ds(sid * _RPT, _RPT)])

    return count_dst


def _gather_combine(a, b, dst2, src2):
    return _sc_gather_combine()(a, b, dst2, src2)


def _scatter_msg(z, dst2, ac):
    return _sc_scatter_msg()(z, dst2, ac)


def _count_dst(dst2):
    return _sc_count_dst()(dst2)


def _lin_stats(e, w, b, g=None):
    n, de = e.shape
    h = w.shape[1]
    nb = n // _BR

    def body(*refs):
        if g is None:
            e_ref, w_ref, b_ref, z_ref, s_ref = refs
            z = jnp.dot(e_ref[...], w_ref[...],
                        preferred_element_type=_F32) + b_ref[...]
        else:
            g_ref, e_ref, w_ref, b_ref, z_ref, s_ref = refs
            z = g_ref[...] + jnp.dot(e_ref[...], w_ref[...],
                                     preferred_element_type=_F32) + b_ref[...]
        z_ref[...] = z

        @pl.when(pl.program_id(0) == 0)
        def _init():
            s_ref[...] = jnp.zeros_like(s_ref)

        s_ref[...] += jnp.stack([jnp.sum(z, 0), jnp.sum(z * z, 0)])

    in_specs = [
        pl.BlockSpec((_BR, de), lambda i: (i, 0)),
        pl.BlockSpec((de, h), lambda i: (0, 0)),
        pl.BlockSpec((1, h), lambda i: (0, 0)),
    ]
    args = [e, w, b]
    if g is not None:
        in_specs.insert(0, pl.BlockSpec((_BR, h), lambda i: (i, 0)))
        args.insert(0, g)
    return pl.pallas_call(
        body,
        grid=(nb,),
        in_specs=in_specs,
        out_specs=[pl.BlockSpec((_BR, h), lambda i: (i, 0)),
                   pl.BlockSpec((2, h), lambda i: (0, 0))],
        out_shape=[jax.ShapeDtypeStruct((n, h), _F32),
                   jax.ShapeDtypeStruct((2, h), _F32)],
    )(*args)


def _act_lin(z1, vec, w2, with_stats):
    n, h = z1.shape
    dout = w2.shape[1]
    nb = n // _BR

    def body(z_ref, v_ref, w_ref, *orefs):
        a = v_ref[0:1, :]
        c = v_ref[1:2, :]
        alpha = v_ref[2:3, :]
        b2 = v_ref[3:4, :dout]
        t = z_ref[...] * a + c
        t = jnp.where(t >= 0.0, t, alpha * t)
        z2 = jnp.dot(t, w_ref[...], preferred_element_type=_F32) + b2
        orefs[0][...] = z2
        if with_stats:
            @pl.when(pl.program_id(0) == 0)
            def _init():
                orefs[1][...] = jnp.zeros_like(orefs[1])
            orefs[1][...] += jnp.stack([jnp.sum(z2, 0), jnp.sum(z2 * z2, 0)])

    out_specs = [pl.BlockSpec((_BR, dout), lambda i: (i, 0))]
    out_shape = [jax.ShapeDtypeStruct((n, dout), _F32)]
    if with_stats:
        out_specs.append(pl.BlockSpec((2, dout), lambda i: (0, 0)))
        out_shape.append(jax.ShapeDtypeStruct((2, dout), _F32))
    res = pl.pallas_call(
        body,
        grid=(nb,),
        in_specs=[pl.BlockSpec((_BR, h), lambda i: (i, 0)),
                  pl.BlockSpec((4, h), lambda i: (0, 0)),
                  pl.BlockSpec((h, dout), lambda i: (0, 0))],
        out_specs=out_specs,
        out_shape=out_shape,
    )(z1, vec, w2)
    return res if with_stats else res[0]


def _act_residual(z, vec, base):
    n, h = z.shape
    nb = n // _BR

    def body(*refs):
        if base is None:
            z_ref, v_ref, o_ref = refs
        else:
            z_ref, v_ref, b_ref, o_ref = refs
        t = z_ref[...] * v_ref[0:1, :] + v_ref[1:2, :]
        t = jnp.where(t >= 0.0, t, v_ref[2:3, :] * t)
        if base is not None:
            t = b_ref[...] + t
        o_ref[...] = t

    in_specs = [pl.BlockSpec((_BR, h), lambda i: (i, 0)),
                pl.BlockSpec((3, h), lambda i: (0, 0))]
    args = [z, vec]
    if base is not None:
        in_specs.append(pl.BlockSpec((_BR, h), lambda i: (i, 0)))
        args.append(base)
    return pl.pallas_call(
        body,
        grid=(nb,),
        in_specs=in_specs,
        out_specs=pl.BlockSpec((_BR, h), lambda i: (i, 0)),
        out_shape=jax.ShapeDtypeStruct((n, h), _F32),
    )(*args)


def _mlp2_block(x, w1, vec, w2):
    n, din = x.shape
    h = w1.shape[1]

    def bn_act(z, gm, bt, al):
        m = jnp.mean(z, 0, keepdims=True)
        v = jnp.mean(z * z, 0, keepdims=True) - m * m
        t = (z - m) * lax.rsqrt(v + _EPS) * gm + bt
        return jnp.where(t >= 0.0, t, al * t)

    def body(x_ref, w1_ref, v_ref, w2_ref, o_ref):
        z1 = jnp.dot(x_ref[...], w1_ref[...],
                     preferred_element_type=_F32) + v_ref[0:1, :]
        t = bn_act(z1, v_ref[1:2, :], v_ref[2:3, :], v_ref[3:4, :])
        z2 = jnp.dot(t, w2_ref[...],
                     preferred_element_type=_F32) + v_ref[4:5, :]
        o_ref[...] = bn_act(z2, v_ref[5:6, :], v_ref[6:7, :], v_ref[7:8, :])

    return pl.pallas_call(
        body,
        in_specs=[pl.BlockSpec((n, din), lambda: (0, 0)),
                  pl.BlockSpec((din, h), lambda: (0, 0)),
                  pl.BlockSpec((8, h), lambda: (0, 0)),
                  pl.BlockSpec((h, h), lambda: (0, 0))],
        out_specs=pl.BlockSpec((n, h), lambda: (0, 0)),
        out_shape=jax.ShapeDtypeStruct((n, h), _F32),
    )(x, w1, vec, w2)


def _node_update(node, accp, cntp, w1a, w1b, vec, w2, wproj, signs):
    n, h = node.shape
    pw = wproj.shape[1]

    def bn_act(z, gm, bt, al):
        m = jnp.mean(z, 0, keepdims=True)
        v = jnp.mean(z * z, 0, keepdims=True) - m * m
        t = (z - m) * lax.rsqrt(v + _EPS) * gm + bt
        return jnp.where(t >= 0.0, t, al * t)

    def body(nd_ref, ac_ref, ct_ref, wa_ref, wb_ref, v_ref, w2_ref,
             wp_ref, sg_ref, on_ref, op_ref):
        cnt = ct_ref[0] + ct_ref[1]
        inv = 1.0 / jnp.maximum(cnt[:, 0:1], 1.0)
        agg = (ac_ref[0] + ac_ref[1]) * inv
        nd = nd_ref[...]
        z1 = (jnp.dot(nd, wa_ref[...], preferred_element_type=_F32)
              + jnp.dot(agg, wb_ref[...], preferred_element_type=_F32)
              + v_ref[0:1, :])
        t = bn_act(z1, v_ref[1:2, :], v_ref[2:3, :], v_ref[3:4, :])
        z2 = jnp.dot(t, w2_ref[...],
                     preferred_element_type=_F32) + v_ref[4:5, :]
        nd_new = nd + bn_act(z2, v_ref[5:6, :], v_ref[6:7, :], v_ref[7:8, :])
        on_ref[...] = nd_new
        op_ref[...] = jnp.dot(nd_new, wp_ref[...],
                              preferred_element_type=_F32) * sg_ref[...]

    return pl.pallas_call(
        body,
        in_specs=[pl.BlockSpec((n, h), lambda: (0, 0)),
                  pl.BlockSpec((2, n, h), lambda: (0, 0, 0)),
                  pl.BlockSpec((2, n, 16), lambda: (0, 0, 0)),
                  pl.BlockSpec((h, h), lambda: (0, 0)),
                  pl.BlockSpec((h, h), lambda: (0, 0)),
                  pl.BlockSpec((8, h), lambda: (0, 0)),
                  pl.BlockSpec((h, h), lambda: (0, 0)),
                  pl.BlockSpec((h, pw), lambda: (0, 0)),
                  pl.BlockSpec((1, pw), lambda: (0, 0))],
        out_specs=[pl.BlockSpec((n, h), lambda: (0, 0)),
                   pl.BlockSpec((n, pw), lambda: (0, 0))],
        out_shape=[jax.ShapeDtypeStruct((n, h), _F32),
                   jax.ShapeDtypeStruct((n, pw), _F32)],
    )(node, accp, cntp, w1a, w1b, vec, w2, wproj, signs)


def _proj_call(node, wproj):
    n, h = node.shape
    pw = wproj.shape[1]

    def body(nd_ref, wp_ref, o_ref):
        o_ref[...] = jnp.dot(nd_ref[...], wp_ref[...],
                             preferred_element_type=_F32)

    return pl.pallas_call(
        body,
        in_specs=[pl.BlockSpec((n, h), lambda: (0, 0)),
                  pl.BlockSpec((h, pw), lambda: (0, 0))],
        out_specs=pl.BlockSpec((n, pw), lambda: (0, 0)),
        out_shape=jax.ShapeDtypeStruct((n, pw), _F32),
    )(node, wproj)


def _fold_bn(stats, n, bn, alpha, b2=None):
    m = stats[0] / n
    v = stats[1] / n - m * m
    a = bn["gamma"] * lax.rsqrt(v + _EPS)
    c = bn["beta"] - m * a
    h = a.shape[0]
    al = jnp.full((h,), alpha, _F32)
    if b2 is None:
        return jnp.stack([a, c, al])
    b2p = jnp.zeros((h,), _F32).at[: b2.shape[0]].set(b2)
    return jnp.stack([a, c, al, b2p])


def _mlp_vec(p):
    return jnp.stack([
        p["lin1"]["b"], p["bn1"]["gamma"], p["bn1"]["beta"],
        jnp.full((_H,), p["pr1"]["alpha"], _F32),
        p["lin2"]["b"], p["bn2"]["gamma"], p["bn2"]["beta"],
        jnp.full((_H,), p["pr2"]["alpha"], _F32),
    ])


def kernel(x, edge_index, edge_attr, params):
    src = edge_index[0]
    dst = edge_index[1]
    dst2 = dst.reshape(_E // _CH, _CH)
    src2 = src.reshape(_E // _CH, _CH)
    layers = params["layers"]
    fin = params["final"]

    pe = params["node_emb"]
    node = _mlp2_block(x, pe["lin1"]["w"], _mlp_vec(pe), pe["lin2"]["w"])

    wm1 = layers[0]["msg"]["lin1"]["w"]
    proj = _proj_call(node, jnp.concatenate([wm1[:_H], wm1[_H:2 * _H]], 1))

    pg = params["edge_emb"]
    z1, s1 = _lin_stats(edge_attr, pg["lin1"]["w"], pg["lin1"]["b"][None])
    v1 = _fold_bn(s1, _E, pg["bn1"], pg["pr1"]["alpha"], pg["lin2"]["b"])
    z2, s2 = _act_lin(z1, v1, pg["lin2"]["w"], True)
    edge = _act_residual(z2, _fold_bn(s2, _E, pg["bn2"], pg["pr2"]["alpha"]),
                         None)

    cntp = _count_dst(dst2)

    for li, lp in enumerate(layers):
        mp, np_, ep = lp["msg"], lp["node"], lp["edge"]

        g1 = _gather_combine(proj[:, :_H], proj[:, _H:2 * _H], dst2, src2)
        wm = mp["lin1"]["w"]
        z1, s1 = _lin_stats(edge, wm[2 * _H:], mp["lin1"]["b"][None], g=g1)
        v1 = _fold_bn(s1, _E, mp["bn1"], mp["pr1"]["alpha"], mp["lin2"]["b"])
        z2, s2 = _act_lin(z1, v1, mp["lin2"]["w"], True)
        ac = _fold_bn(s2, _E, mp["bn2"], mp["pr2"]["alpha"])
        accp = _scatter_msg(z2, dst2, ac)

        we = ep["lin1"]["w"][_H:]
        if li + 1 < len(layers):
            wn = layers[li + 1]["msg"]["lin1"]["w"]
            wp = jnp.concatenate([we, we, wn[:_H], wn[_H:2 * _H]], 1)
            signs = jnp.concatenate([
                jnp.full((1, _H), -1.0, _F32), jnp.full((1, _H), 1.0, _F32),
                jnp.full((1, 2 * _H), 1.0, _F32)], 1)
        else:
            wf = fin["lin1"]["w"][:_H]
            wp = jnp.concatenate([we, we, wf, wf], 1)
            signs = jnp.concatenate([
                jnp.full((1, _H), -1.0, _F32), jnp.full((1, _H), 1.0, _F32),
                jnp.full((1, _H), 1.0, _F32), jnp.full((1, _H), -1.0, _F32)],
                1)
        wn1 = np_["lin1"]["w"]
        node, proj4 = _node_update(node, accp, cntp, wn1[:_H], wn1[_H:],
                                   _mlp_vec(np_), np_["lin2"]["w"], wp, signs)

        d = _gather_combine(proj4[:, :_H], proj4[:, _H:2 * _H], dst2, src2)
        z1, s1 = _lin_stats(edge, ep["lin1"]["w"][:_H], ep["lin1"]["b"][None],
                            g=d)
        v1 = _fold_bn(s1, _E, ep["bn1"], ep["pr1"]["alpha"], ep["lin2"]["b"])
        z2, s2 = _act_lin(z1, v1, ep["lin2"]["w"], True)
        edge = _act_residual(
            z2, _fold_bn(s2, _E, ep["bn2"], ep["pr2"]["alpha"]), edge)
        proj = proj4[:, 2 * _H:]

    gf = _gather_combine(proj[:, :_H], proj[:, _H:], dst2, src2)
    z1, s1 = _lin_stats(edge, fin["lin1"]["w"][_H:], fin["lin1"]["b"][None],
                        g=gf)
    v1 = _fold_bn(s1, _E, fin["bn1"], fin["pr1"]["alpha"], fin["lin2"]["b"])
    out = _act_lin(z1, v1, fin["lin2"]["w"], False)
    return out


# speedup vs baseline: 3.6939x; 2.1018x over previous
import functools

import jax
import jax.numpy as jnp
from jax import lax
from jax.experimental import pallas as pl
from jax.experimental.pallas import tpu as pltpu
from jax.experimental.pallas import tpu_sc as plsc
from jax.scipy.linalg import block_diag

_N = 10000
_E = 320000
_H = 32
_EPS = 1e-5

_NC = 2
_NS = 16
_NW = _NC * _NS
_EPW = _E // _NW
_CH = 80
_CH4 = _CH // 4
_NCH = _EPW // _CH
_RPT = _N // _NS

_E4 = _E // 4
_BR4 = 4000
_F32 = jnp.float32


@functools.cache
def _sc_mesh():
    return plsc.VectorSubcoreMesh(core_axis_name="c", subcore_axis_name="s",
                                  num_cores=_NC, num_subcores=_NS)


@functools.cache
def _sc_gather_combine():
    @functools.partial(
        pl.kernel,
        out_type=jax.ShapeDtypeStruct((_E4, 128), _F32),
        mesh=_sc_mesh(),
        compiler_params=pltpu.CompilerParams(use_tc_tiling_on_sc=False),
        scratch_types=[
            pltpu.VMEM((_NCH, _CH), jnp.int32),
            pltpu.VMEM((_NCH, _CH), jnp.int32),
            pltpu.VMEM((_CH, _H), _F32),
            pltpu.VMEM((_CH, _H), _F32),
            pltpu.VMEM((_CH, _H), _F32),
            pltpu.VMEM((_CH, _H), _F32),
            pltpu.VMEM((_CH4, 128), _F32),
            pltpu.VMEM((_CH4, 128), _F32),
            pltpu.SemaphoreType.DMA,
            pltpu.SemaphoreType.DMA,
        ],
    )
    def gather_combine(a_hbm, b_hbm, dst2_hbm, src2_hbm, out_hbm,
                       idx_a, idx_b, ra0, rb0, ra1, rb1, sb0, sb1,
                       sem0, sem1):
        wid = lax.axis_index("s") * _NC + lax.axis_index("c")
        wrow = wid * _NCH
        wrow4 = wid * (_EPW // 4)
        pltpu.sync_copy(dst2_hbm.at[pl.ds(wrow, _NCH)], idx_a)
        pltpu.sync_copy(src2_hbm.at[pl.ds(wrow, _NCH)], idx_b)

        def fire(c, ra_, rb_, sem_):
            pltpu.async_copy(a_hbm.at[idx_a.at[c]], ra_, sem_)
            pltpu.async_copy(b_hbm.at[idx_b.at[c]], rb_, sem_)

        def drain_process(c, ra_, rb_, sb_, sem_):
            pltpu.make_async_copy(a_hbm.at[idx_a.at[c]], ra_, sem_).wait()
            pltpu.make_async_copy(b_hbm.at[idx_b.at[c]], rb_, sem_).wait()

            def addrow(r, c2):
                q = r // 4
                o = (r % 4) * _H
                for h in range(2):
                    sl = pl.ds(h * 16, 16)
                    sb_[q, pl.ds(o + h * 16, 16)] = ra_[r, sl] + rb_[r, sl]
                return c2

            lax.fori_loop(0, _CH, addrow, 0, unroll=4)
            pltpu.sync_copy(sb_, out_hbm.at[pl.ds(wrow4 + c * _CH4, _CH4)])

        fire(0, ra0, rb0, sem0)

        def pair(p, carry):
            c0 = 2 * p
            fire(c0 + 1, ra1, rb1, sem1)
            drain_process(c0, ra0, rb0, sb0, sem0)

            @pl.when(c0 + 2 < _NCH)
            def _():
                fire(c0 + 2, ra0, rb0, sem0)

            drain_process(c0 + 1, ra1, rb1, sb1, sem1)
            return carry

        lax.fori_loop(0, _NCH // 2, pair, 0)
        drain_process(_NCH - 1, ra0, rb0, sb0, sem0)

    return gather_combine


@functools.cache
def _sc_scatter_msg():
    @functools.partial(
        pl.kernel,
        out_type=jax.ShapeDtypeStruct((_NC, _N, _H), _F32),
        mesh=_sc_mesh(),
        compiler_params=pltpu.CompilerParams(use_tc_tiling_on_sc=False),
        scratch_types=[
            pltpu.VMEM((_NCH, _CH), jnp.int32),
            pltpu.VMEM((_CH4, 128), _F32),
            pltpu.VMEM((_CH4, 128), _F32),
            pltpu.VMEM((_CH, _H), _F32),
            pltpu.VMEM((_CH, _H), _F32),
            pltpu.VMEM((_RPT, _H), _F32),
            pltpu.VMEM((3, _H), _F32),
            pltpu.VMEM_SHARED((_N, _H), _F32),
            pltpu.SemaphoreType.DMA,
            pltpu.SemaphoreType.DMA,
            pltpu.SemaphoreType.DMA,
            pltpu.SemaphoreType.DMA,
        ],
    )
    def scatter_msg(z_hbm, dst2_hbm, ac_hbm, out_hbm,
                    idx_all, zp0, zp1, zs0, zs1, tbuf, acv, acc,
                    semz0, semz1, sems0, sems1):
        cid = lax.axis_index("c")
        sid = lax.axis_index("s")
        wid = sid * _NC + cid
        wrow = wid * _NCH
        wrow4 = wid * (_EPW // 4)

        def zrow(r, carry):
            for h in range(2):
                tbuf[r, pl.ds(h * 16, 16)] = jnp.zeros((16,), _F32)
            return carry

        lax.fori_loop(0, _RPT, zrow, 0)
        pltpu.sync_copy(tbuf, acc.at[pl.ds(sid * _RPT, _RPT)])
        pltpu.sync_copy(dst2_hbm.at[pl.ds(wrow, _NCH)], idx_all)
        plsc.subcore_barrier()

        pltpu.sync_copy(ac_hbm, acv)
        a_lo = acv[0, pl.ds(0, 16)]
        a_hi = acv[0, pl.ds(16, 16)]
        c_lo = acv[1, pl.ds(0, 16)]
        c_hi = acv[1, pl.ds(16, 16)]
        p_lo = acv[2, pl.ds(0, 16)]
        p_hi = acv[2, pl.ds(16, 16)]

        def fire_z(c, zp_, semz_):
            pltpu.async_copy(z_hbm.at[pl.ds(wrow4 + c * _CH4, _CH4)], zp_,
                             semz_)

        def process(c, zp_, zs_, semz_, sems_):
            pltpu.make_async_copy(
                z_hbm.at[pl.ds(wrow4 + c * _CH4, _CH4)], zp_, semz_).wait()

            def prow(r, c2):
                q = r // 4
                o = (r % 4) * _H
                for h, (av, cv, pv) in enumerate(((a_lo, c_lo, p_lo),
                                                  (a_hi, c_hi, p_hi))):
                    v = zp_[q, pl.ds(o + h * 16, 16)] * av + cv
                    zs_[r, pl.ds(h * 16, 16)] = jnp.where(v >= 0.0, v, pv * v)
                return c2

            lax.fori_loop(0, _CH, prow, 0, unroll=4)
            pltpu.async_copy(zs_, acc.at[idx_all.at[c]], sems_, add=True)

        def wait_scat(c, zs_, sems_):
            pltpu.make_async_copy(zs_, acc.at[idx_all.at[c]], sems_).wait()

        fire_z(0, zp0, semz0)
        fire_z(1, zp1, semz1)

        def pair(p, carry):
            c0 = 2 * p
            process(c0, zp0, zs0, semz0, sems0)

            @pl.when(c0 + 2 < _NCH)
            def _():
                wait_scat(c0, zs0, sems0)
                fire_z(c0 + 2, zp0, semz0)

            process(c0 + 1, zp1, zs1, semz1, sems1)

            @pl.when(c0 + 3 < _NCH)
            def _():
                wait_scat(c0 + 1, zs1, sems1)
                fire_z(c0 + 3, zp1, semz1)

            return carry

        lax.fori_loop(0, _NCH // 2, pair, 0)
        process(_NCH - 1, zp0, zs0, semz0, sems0)
        wait_scat(_NCH - 1, zs0, sems0)
        wait_scat(_NCH - 2, zs1, sems1)
        plsc.subcore_barrier()

        pltpu.sync_copy(acc.at[pl.ds(sid * _RPT, _RPT)], tbuf)
        pltpu.sync_copy(tbuf, out_hbm.at[cid, pl.ds(sid * _RPT, _RPT)])

    return scatter_msg


@functools.cache
def _sc_count_dst():
    @functools.partial(
        pl.kernel,
        out_type=jax.ShapeDtypeStruct((_NC, _N, 16), _F32),
        mesh=_sc_mesh(),
        compiler_params=pltpu.CompilerParams(use_tc_tiling_on_sc=False),
        scratch_types=[
            pltpu.VMEM((_NCH, _CH), jnp.int32),
            pltpu.VMEM((_CH, 16), _F32),
            pltpu.VMEM((_RPT, 16), _F32),
            pltpu.VMEM_SHARED((_N, 16), _F32),
            pltpu.SemaphoreType.DMA,
        ],
    )
    def count_dst(dst2_hbm, out_hbm, idx_all, ones, tbuf, acc, sem):
        cid = lax.axis_index("c")
        sid = lax.axis_index("s")
        wid = sid * _NC + cid
        wrow = wid * _NCH

        def zrow(r, carry):
            tbuf[r, pl.ds(0, 16)] = jnp.zeros((16,), _F32)
            return carry

        lax.fori_loop(0, _RPT, zrow, 0)
        pltpu.sync_copy(tbuf, acc.at[pl.ds(sid * _RPT, _RPT)])
        pltpu.sync_copy(dst2_hbm.at[pl.ds(wrow, _NCH)], idx_all)

        lane = lax.broadcasted_iota(jnp.int32, (16,), 0)
        one_row = jnp.where(lane == 0, 1.0, 0.0).astype(_F32)

        def orow(r, carry):
            ones[r, pl.ds(0, 16)] = one_row
            return carry

        lax.fori_loop(0, _CH, orow, 0)
        plsc.subcore_barrier()

        def chunk(c, carry):
            pltpu.async_copy(ones, acc.at[idx_all.at[c]], sem, add=True)
            return carry

        lax.fori_loop(0, _NCH, chunk, 0)

        def drain(c, carry):
            pltpu.make_async_copy(ones, acc.at[idx_all.at[c]], sem).wait()
            return carry

        lax.fori_loop(0, _NCH, drain, 0)
        plsc.subcore_barrier()

        pltpu.sync_copy(acc.at[pl.ds(sid * _RPT, _RPT)], tbuf)
        pltpu.sync_copy(tbuf, out_hbm.at[cid, pl.ds(sid * _RPT, _RPT)])

    return count_dst


def _gather_combine(a, b, dst2, src2):
    return _sc_gather_combine()(a, b, dst2, src2)


def _scatter_msg(z, dst2, ac):
    return _sc_scatter_msg()(z, dst2, ac)


def _count_dst(dst2):
    return _sc_count_dst()(dst2)


def _lin_stats(e, w, b, g=None):
    n4, dp = e.shape

    def body(*refs):
        if g is None:
            e_ref, w_ref, b_ref, z_ref, s_ref = refs
            z = jnp.dot(e_ref[...], w_ref[...],
                        preferred_element_type=_F32) + b_ref[...]
        else:
            g_ref, e_ref, w_ref, b_ref, z_ref, s_ref = refs
            z = g_ref[...] + jnp.dot(e_ref[...], w_ref[...],
                                     preferred_element_type=_F32) + b_ref[...]
        z_ref[...] = z

        @pl.when(pl.program_id(0) == 0)
        def _init():
            s_ref[...] = jnp.zeros_like(s_ref)

        s_ref[...] += jnp.stack([jnp.sum(z, 0), jnp.sum(z * z, 0)])

    in_specs = [
        pl.BlockSpec((_BR4, dp), lambda i: (i, 0)),
        pl.BlockSpec((dp, 128), lambda i: (0, 0)),
        pl.BlockSpec((1, 128), lambda i: (0, 0)),
    ]
    args = [e, w, b]
    if g is not None:
        in_specs.insert(0, pl.BlockSpec((_BR4, 128), lambda i: (i, 0)))
        args.insert(0, g)
    return pl.pallas_call(
        body,
        grid=(n4 // _BR4,),
        in_specs=in_specs,
        out_specs=[pl.BlockSpec((_BR4, 128), lambda i: (i, 0)),
                   pl.BlockSpec((2, 128), lambda i: (0, 0))],
        out_shape=[jax.ShapeDtypeStruct((n4, 128), _F32),
                   jax.ShapeDtypeStruct((2, 128), _F32)],
    )(*args)


def _act_lin(z1, vec, w2, with_stats):
    n4 = z1.shape[0]
    dout = w2.shape[1]

    def body(z_ref, v_ref, w_ref, *orefs):
        t = z_ref[...] * v_ref[0:1, :] + v_ref[1:2, :]
        t = jnp.where(t >= 0.0, t, v_ref[2:3, :] * t)
        z2 = jnp.dot(t, w_ref[...],
                     preferred_element_type=_F32) + v_ref[3:4, :dout]
        orefs[0][...] = z2
        if with_stats:
            @pl.when(pl.program_id(0) == 0)
            def _init():
                orefs[1][...] = jnp.zeros_like(orefs[1])
            orefs[1][...] += jnp.stack([jnp.sum(z2, 0), jnp.sum(z2 * z2, 0)])

    out_specs = [pl.BlockSpec((_BR4, dout), lambda i: (i, 0))]
    out_shape = [jax.ShapeDtypeStruct((n4, dout), _F32)]
    if with_stats:
        out_specs.append(pl.BlockSpec((2, dout), lambda i: (0, 0)))
        out_shape.append(jax.ShapeDtypeStruct((2, dout), _F32))
    res = pl.pallas_call(
        body,
        grid=(n4 // _BR4,),
        in_specs=[pl.BlockSpec((_BR4, 128), lambda i: (i, 0)),
                  pl.BlockSpec((4, 128), lambda i: (0, 0)),
                  pl.BlockSpec((128, dout), lambda i: (0, 0))],
        out_specs=out_specs,
        out_shape=out_shape,
    )(z1, vec, w2)
    return res if with_stats else res[0]


def _act_residual(z, vec, base):
    n4 = z.shape[0]

    def body(*refs):
        if base is None:
            z_ref, v_ref, o_ref = refs
        else:
            z_ref, v_ref, b_ref, o_ref = refs
        t = z_ref[...] * v_ref[0:1, :] + v_ref[1:2, :]
        t = jnp.where(t >= 0.0, t, v_ref[2:3, :] * t)
        if base is not None:
            t = b_ref[...] + t
        o_ref[...] = t

    in_specs = [pl.BlockSpec((_BR4, 128), lambda i: (i, 0)),
                pl.BlockSpec((3, 128), lambda i: (0, 0))]
    args = [z, vec]
    if base is not None:
        in_specs.append(pl.BlockSpec((_BR4, 128), lambda i: (i, 0)))
        args.append(base)
    return pl.pallas_call(
        body,
        grid=(n4 // _BR4,),
        in_specs=in_specs,
        out_specs=pl.BlockSpec((_BR4, 128), lambda i: (i, 0)),
        out_shape=jax.ShapeDtypeStruct((n4, 128), _F32),
    )(*args)


def _mlp2_block(x, w1, vec, w2):
    n, din = x.shape
    h = w1.shape[1]

    def bn_act(z, gm, bt, al):
        m = jnp.mean(z, 0, keepdims=True)
        v = jnp.mean(z * z, 0, keepdims=True) - m * m
        t = (z - m) * lax.rsqrt(v + _EPS) * gm + bt
        return jnp.where(t >= 0.0, t, al * t)

    def body(x_ref, w1_ref, v_ref, w2_ref, o_ref):
        z1 = jnp.dot(x_ref[...], w1_ref[...],
                     preferred_element_type=_F32) + v_ref[0:1, :]
        t = bn_act(z1, v_ref[1:2, :], v_ref[2:3, :], v_ref[3:4, :])
        z2 = jnp.dot(t, w2_ref[...],
                     preferred_element_type=_F32) + v_ref[4:5, :]
        o_ref[...] = bn_act(z2, v_ref[5:6, :], v_ref[6:7, :], v_ref[7:8, :])

    return pl.pallas_call(
        body,
        in_specs=[pl.BlockSpec((n, din), lambda: (0, 0)),
                  pl.BlockSpec((din, h), lambda: (0, 0)),
                  pl.BlockSpec((8, h), lambda: (0, 0)),
                  pl.BlockSpec((h, h), lambda: (0, 0))],
        out_specs=pl.BlockSpec((n, h), lambda: (0, 0)),
        out_shape=jax.ShapeDtypeStruct((n, h), _F32),
    )(x, w1, vec, w2)


def _node_update(node, accp, cntp, w1a, w1b, vec, w2, wproj, signs):
    n, h = node.shape
    pw = wproj.shape[1]

    def bn_act(z, gm, bt, al):
        m = jnp.mean(z, 0, keepdims=True)
        v = jnp.mean(z * z, 0, keepdims=True) - m * m
        t = (z - m) * lax.rsqrt(v + _EPS) * gm + bt
        return jnp.where(t >= 0.0, t, al * t)

    def body(nd_ref, ac_ref, ct_ref, wa_ref, wb_ref, v_ref, w2_ref,
             wp_ref, sg_ref, on_ref, op_ref):
        cnt = ct_ref[0] + ct_ref[1]
        inv = 1.0 / jnp.maximum(cnt[:, 0:1], 1.0)
        agg = (ac_ref[0] + ac_ref[1]) * inv
        nd = nd_ref[...]
        z1 = (jnp.dot(nd, wa_ref[...], preferred_element_type=_F32)
              + jnp.dot(agg, wb_ref[...], preferred_element_type=_F32)
              + v_ref[0:1, :])
        t = bn_act(z1, v_ref[1:2, :], v_ref[2:3, :], v_ref[3:4, :])
        z2 = jnp.dot(t, w2_ref[...],
                     preferred_element_type=_F32) + v_ref[4:5, :]
        nd_new = nd + bn_act(z2, v_ref[5:6, :], v_ref[6:7, :], v_ref[7:8, :])
        on_ref[...] = nd_new
        op_ref[...] = jnp.dot(nd_new, wp_ref[...],
                              preferred_element_type=_F32) * sg_ref[...]

    return pl.pallas_call(
        body,
        in_specs=[pl.BlockSpec((n, h), lambda: (0, 0)),
                  pl.BlockSpec((2, n, h), lambda: (0, 0, 0)),
                  pl.BlockSpec((2, n, 16), lambda: (0, 0, 0)),
                  pl.BlockSpec((h, h), lambda: (0, 0)),
                  pl.BlockSpec((h, h), lambda: (0, 0)),
                  pl.BlockSpec((8, h), lambda: (0, 0)),
                  pl.BlockSpec((h, h), lambda: (0, 0)),
                  pl.BlockSpec((h, pw), lambda: (0, 0)),
                  pl.BlockSpec((1, pw), lambda: (0, 0))],
        out_specs=[pl.BlockSpec((n, h), lambda: (0, 0)),
                   pl.BlockSpec((n, pw), lambda: (0, 0))],
        out_shape=[jax.ShapeDtypeStruct((n, h), _F32),
                   jax.ShapeDtypeStruct((n, pw), _F32)],
    )(node, accp, cntp, w1a, w1b, vec, w2, wproj, signs)


def _proj_call(node, wproj):
    n, h = node.shape
    pw = wproj.shape[1]

    def body(nd_ref, wp_ref, o_ref):
        o_ref[...] = jnp.dot(nd_ref[...], wp_ref[...],
                             preferred_element_type=_F32)

    return pl.pallas_call(
        body,
        in_specs=[pl.BlockSpec((n, h), lambda: (0, 0)),
                  pl.BlockSpec((h, pw), lambda: (0, 0))],
        out_specs=pl.BlockSpec((n, pw), lambda: (0, 0)),
        out_shape=jax.ShapeDtypeStruct((n, pw), _F32),
    )(node, wproj)


def _bd4(w):
    return block_diag(w, w, w, w)


def _t4(v):
    return jnp.tile(v, 4)


def _fold_bn(stats, n, bn, alpha):
    s = stats.reshape(2, 4, _H).sum(axis=1)
    m = s[0] / n
    v = s[1] / n - m * m
    a = bn["gamma"] * lax.rsqrt(v + _EPS)
    c = bn["beta"] - m * a
    return jnp.stack([a, c, jnp.full((_H,), alpha, _F32)])


def _vec4(ac3, b2, dout):
    row_b = jnp.zeros((128,), _F32).at[: 4 * dout].set(jnp.tile(b2, 4))
    return jnp.concatenate([jnp.tile(ac3, (1, 4)), row_b[None]], axis=0)


def _mlp_vec(p):
    return jnp.stack([
        p["lin1"]["b"], p["bn1"]["gamma"], p["bn1"]["beta"],
        jnp.full((_H,), p["pr1"]["alpha"], _F32),
        p["lin2"]["b"], p["bn2"]["gamma"], p["bn2"]["beta"],
        jnp.full((_H,), p["pr2"]["alpha"], _F32),
    ])


def kernel(x, edge_index, edge_attr, params):
    src = edge_index[0]
    dst = edge_index[1]
    dst2 = dst.reshape(_E // _CH, _CH)
    src2 = src.reshape(_E // _CH, _CH)
    layers = params["layers"]
    fin = params["final"]

    pe = params["node_emb"]
    node = _mlp2_block(x, pe["lin1"]["w"], _mlp_vec(pe), pe["lin2"]["w"])

    wm1 = layers[0]["msg"]["lin1"]["w"]
    proj = _proj_call(node, jnp.concatenate([wm1[:_H], wm1[_H:2 * _H]], 1))

    pg = params["edge_emb"]
    ea4 = edge_attr.reshape(_E4, 64)
    z1, s1 = _lin_stats(ea4, _bd4(pg["lin1"]["w"]),
                        _t4(pg["lin1"]["b"])[None])
    v1 = _fold_bn(s1, _E, pg["bn1"], pg["pr1"]["alpha"])
    z2, s2 = _act_lin(z1, _vec4(v1, pg["lin2"]["b"], _H),
                      _bd4(pg["lin2"]["w"]), True)
    v2 = _fold_bn(s2, _E, pg["bn2"], pg["pr2"]["alpha"])
    edge = _act_residual(z2, jnp.tile(v2, (1, 4)), None)

    cntp = _count_dst(dst2)

    for li, lp in enumerate(layers):
        mp, np_, ep = lp["msg"], lp["node"], lp["edge"]

        g1 = _gather_combine(proj[:, :_H], proj[:, _H:2 * _H], dst2, src2)
        wm = mp["lin1"]["w"]
        z1, s1 = _lin_stats(edge, _bd4(wm[2 * _H:]),
                            _t4(mp["lin1"]["b"])[None], g=g1)
        v1 = _fold_bn(s1, _E, mp["bn1"], mp["pr1"]["alpha"])
        z2, s2 = _act_lin(z1, _vec4(v1, mp["lin2"]["b"], _H),
                          _bd4(mp["lin2"]["w"]), True)
        ac = _fold_bn(s2, _E, mp["bn2"], mp["pr2"]["alpha"])
        accp = _scatter_msg(z2, dst2, ac)

        we = ep["lin1"]["w"][_H:]
        if li + 1 < len(layers):
            wn = layers[li + 1]["msg"]["lin1"]["w"]
            wp = jnp.concatenate([we, we, wn[:_H], wn[_H:2 * _H]], 1)
            signs = jnp.concatenate([
                jnp.full((1, _H), -1.0, _F32), jnp.full((1, _H), 1.0, _F32),
                jnp.full((1, 2 * _H), 1.0, _F32)], 1)
        else:
            wf = fin["lin1"]["w"][:_H]
            wp = jnp.concatenate([we, we, wf, wf], 1)
            signs = jnp.concatenate([
                jnp.full((1, _H), -1.0, _F32), jnp.full((1, _H), 1.0, _F32),
                jnp.full((1, _H), 1.0, _F32), jnp.full((1, _H), -1.0, _F32)],
                1)
        wn1 = np_["lin1"]["w"]
        node, proj4 = _node_update(node, accp, cntp, wn1[:_H], wn1[_H:],
                                   _mlp_vec(np_), np_["lin2"]["w"], wp, signs)

        d = _gather_combine(proj4[:, :_H], proj4[:, _H:2 * _H], dst2, src2)
        z1, s1 = _lin_stats(edge, _bd4(ep["lin1"]["w"][:_H]),
                            _t4(ep["lin1"]["b"])[None], g=d)
        v1 = _fold_bn(s1, _E, ep["bn1"], ep["pr1"]["alpha"])
        z2, s2 = _act_lin(z1, _vec4(v1, ep["lin2"]["b"], _H),
                          _bd4(ep["lin2"]["w"]), True)
        v2 = _fold_bn(s2, _E, ep["bn2"], ep["pr2"]["alpha"])
        edge = _act_residual(z2, jnp.tile(v2, (1, 4)), edge)
        proj = proj4[:, 2 * _H:]

    gf = _gather_combine(proj[:, :_H], proj[:, _H:], dst2, src2)
    z1, s1 = _lin_stats(edge, _bd4(fin["lin1"]["w"][_H:]),
                        _t4(fin["lin1"]["b"])[None], g=gf)
    v1 = _fold_bn(s1, _E, fin["bn1"], fin["pr1"]["alpha"])
    out4 = _act_lin(z1, _vec4(v1, fin["lin2"]["b"], 3),
                    _bd4(fin["lin2"]["w"]), False)
    return out4.reshape(_E, 3)


# speedup vs baseline: 3.6964x; 1.0007x over previous
import functools

import jax
import jax.numpy as jnp
from jax import lax
from jax.experimental import pallas as pl
from jax.experimental.pallas import tpu as pltpu
from jax.experimental.pallas import tpu_sc as plsc
from jax.scipy.linalg import block_diag

_N = 10000
_E = 320000
_H = 32
_EPS = 1e-5

_NC = 2
_NS = 16
_NW = _NC * _NS
_EPW = _E // _NW
_CH = 80
_CH4 = _CH // 4
_NCH = _EPW // _CH
_RPT = _N // _NS

_E4 = _E // 4
_BR4 = 4000
_F32 = jnp.float32


@functools.cache
def _sc_mesh():
    return plsc.VectorSubcoreMesh(core_axis_name="c", subcore_axis_name="s",
                                  num_cores=_NC, num_subcores=_NS)


@functools.cache
def _sc_gather_combine():
    @functools.partial(
        pl.kernel,
        out_type=jax.ShapeDtypeStruct((_E4, 128), _F32),
        mesh=_sc_mesh(),
        compiler_params=pltpu.CompilerParams(use_tc_tiling_on_sc=False),
        scratch_types=[
            pltpu.VMEM((_NCH, _CH), jnp.int32),
            pltpu.VMEM((_NCH, _CH), jnp.int32),
            pltpu.VMEM((_CH, _H), _F32),
            pltpu.VMEM((_CH, _H), _F32),
            pltpu.VMEM((_CH, _H), _F32),
            pltpu.VMEM((_CH, _H), _F32),
            pltpu.VMEM((_CH4, 128), _F32),
            pltpu.VMEM((_CH4, 128), _F32),
            pltpu.SemaphoreType.DMA,
            pltpu.SemaphoreType.DMA,
        ],
    )
    def gather_combine(a_hbm, b_hbm, dst2_hbm, src2_hbm, out_hbm,
                       idx_a, idx_b, ra0, rb0, ra1, rb1, sb0, sb1,
                       sem0, sem1):
        wid = lax.axis_index("s") * _NC + lax.axis_index("c")
        wrow = wid * _NCH
        wrow4 = wid * (_EPW // 4)
        pltpu.sync_copy(dst2_hbm.at[pl.ds(wrow, _NCH)], idx_a)
        pltpu.sync_copy(src2_hbm.at[pl.ds(wrow, _NCH)], idx_b)

        def fire(c, ra_, rb_, sem_):
            pltpu.async_copy(a_hbm.at[idx_a.at[c]], ra_, sem_)
            pltpu.async_copy(b_hbm.at[idx_b.at[c]], rb_, sem_)

        def drain_process(c, ra_, rb_, sb_, sem_):
            pltpu.make_async_copy(a_hbm.at[idx_a.at[c]], ra_, sem_).wait()
            pltpu.make_async_copy(b_hbm.at[idx_b.at[c]], rb_, sem_).wait()

            def addq(q, c2):
                r0 = q * 4
                for k in range(4):
                    for h in range(2):
                        sl = pl.ds(h * 16, 16)
                        sb_[q, pl.ds(k * _H + h * 16, 16)] = (
                            ra_[r0 + k, sl] + rb_[r0 + k, sl])
                return c2

            lax.fori_loop(0, _CH4, addq, 0, unroll=2)
            pltpu.sync_copy(sb_, out_hbm.at[pl.ds(wrow4 + c * _CH4, _CH4)])

        fire(0, ra0, rb0, sem0)

        def pair(p, carry):
            c0 = 2 * p
            fire(c0 + 1, ra1, rb1, sem1)
            drain_process(c0, ra0, rb0, sb0, sem0)

            @pl.when(c0 + 2 < _NCH)
            def _():
                fire(c0 + 2, ra0, rb0, sem0)

            drain_process(c0 + 1, ra1, rb1, sb1, sem1)
            return carry

        lax.fori_loop(0, _NCH // 2, pair, 0)
        drain_process(_NCH - 1, ra0, rb0, sb0, sem0)

    return gather_combine


@functools.cache
def _sc_scatter_msg():
    @functools.partial(
        pl.kernel,
        out_type=jax.ShapeDtypeStruct((_NC, _N, _H), _F32),
        mesh=_sc_mesh(),
        compiler_params=pltpu.CompilerParams(use_tc_tiling_on_sc=False),
        scratch_types=[
            pltpu.VMEM((_NCH, _CH), jnp.int32),
            pltpu.VMEM((_CH4, 128), _F32),
            pltpu.VMEM((_CH4, 128), _F32),
            pltpu.VMEM((_CH, _H), _F32),
            pltpu.VMEM((_CH, _H), _F32),
            pltpu.VMEM((_RPT, _H), _F32),
            pltpu.VMEM((3, _H), _F32),
            pltpu.VMEM_SHARED((_N, _H), _F32),
            pltpu.SemaphoreType.DMA,
            pltpu.SemaphoreType.DMA,
            pltpu.SemaphoreType.DMA,
            pltpu.SemaphoreType.DMA,
        ],
    )
    def scatter_msg(z_hbm, dst2_hbm, ac_hbm, out_hbm,
                    idx_all, zp0, zp1, zs0, zs1, tbuf, acv, acc,
                    semz0, semz1, sems0, sems1):
        cid = lax.axis_index("c")
        sid = lax.axis_index("s")
        wid = sid * _NC + cid
        wrow = wid * _NCH
        wrow4 = wid * (_EPW // 4)

        def zrow(r, carry):
            for h in range(2):
                tbuf[r, pl.ds(h * 16, 16)] = jnp.zeros((16,), _F32)
            return carry

        lax.fori_loop(0, _RPT, zrow, 0)
        pltpu.sync_copy(tbuf, acc.at[pl.ds(sid * _RPT, _RPT)])
        pltpu.sync_copy(dst2_hbm.at[pl.ds(wrow, _NCH)], idx_all)
        plsc.subcore_barrier()

        pltpu.sync_copy(ac_hbm, acv)
        a_lo = acv[0, pl.ds(0, 16)]
        a_hi = acv[0, pl.ds(16, 16)]
        c_lo = acv[1, pl.ds(0, 16)]
        c_hi = acv[1, pl.ds(16, 16)]
        p_lo = acv[2, pl.ds(0, 16)]
        p_hi = acv[2, pl.ds(16, 16)]

        def fire_z(c, zp_, semz_):
            pltpu.async_copy(z_hbm.at[pl.ds(wrow4 + c * _CH4, _CH4)], zp_,
                             semz_)

        def process(c, zp_, zs_, semz_, sems_):
            pltpu.make_async_copy(
                z_hbm.at[pl.ds(wrow4 + c * _CH4, _CH4)], zp_, semz_).wait()

            def pq(q, c2):
                r0 = q * 4
                for k in range(4):
                    for h, (av, cv, pv) in enumerate(((a_lo, c_lo, p_lo),
                                                      (a_hi, c_hi, p_hi))):
                        v = zp_[q, pl.ds(k * _H + h * 16, 16)] * av + cv
                        zs_[r0 + k, pl.ds(h * 16, 16)] = jnp.where(
                            v >= 0.0, v, pv * v)
                return c2

            lax.fori_loop(0, _CH4, pq, 0, unroll=2)
            pltpu.async_copy(zs_, acc.at[idx_all.at[c]], sems_, add=True)

        def wait_scat(c, zs_, sems_):
            pltpu.make_async_copy(zs_, acc.at[idx_all.at[c]], sems_).wait()

        fire_z(0, zp0, semz0)
        fire_z(1, zp1, semz1)

        def pair(p, carry):
            c0 = 2 * p
            process(c0, zp0, zs0, semz0, sems0)

            @pl.when(c0 + 2 < _NCH)
            def _():
                wait_scat(c0, zs0, sems0)
                fire_z(c0 + 2, zp0, semz0)

            process(c0 + 1, zp1, zs1, semz1, sems1)

            @pl.when(c0 + 3 < _NCH)
            def _():
                wait_scat(c0 + 1, zs1, sems1)
                fire_z(c0 + 3, zp1, semz1)

            return carry

        lax.fori_loop(0, _NCH // 2, pair, 0)
        process(_NCH - 1, zp0, zs0, semz0, sems0)
        wait_scat(_NCH - 1, zs0, sems0)
        wait_scat(_NCH - 2, zs1, sems1)
        plsc.subcore_barrier()

        pltpu.sync_copy(acc.at[pl.ds(sid * _RPT, _RPT)], tbuf)
        pltpu.sync_copy(tbuf, out_hbm.at[cid, pl.ds(sid * _RPT, _RPT)])

    return scatter_msg


@functools.cache
def _sc_count_dst():
    @functools.partial(
        pl.kernel,
        out_type=jax.ShapeDtypeStruct((_NC, _N, 16), _F32),
        mesh=_sc_mesh(),
        compiler_params=pltpu.CompilerParams(use_tc_tiling_on_sc=False),
        scratch_types=[
            pltpu.VMEM((_NCH, _CH), jnp.int32),
            pltpu.VMEM((_CH, 16), _F32),
            pltpu.VMEM((_RPT, 16), _F32),
            pltpu.VMEM_SHARED((_N, 16), _F32),
            pltpu.SemaphoreType.DMA,
        ],
    )
    def count_dst(dst2_hbm, out_hbm, idx_all, ones, tbuf, acc, sem):
        cid = lax.axis_index("c")
        sid = lax.axis_index("s")
        wid = sid * _NC + cid
        wrow = wid * _NCH

        def zrow(r, carry):
            tbuf[r, pl.ds(0, 16)] = jnp.zeros((16,), _F32)
            return carry

        lax.fori_loop(0, _RPT, zrow, 0)
        pltpu.sync_copy(tbuf, acc.at[pl.ds(sid * _RPT, _RPT)])
        pltpu.sync_copy(dst2_hbm.at[pl.ds(wrow, _NCH)], idx_all)

        lane = lax.broadcasted_iota(jnp.int32, (16,), 0)
        one_row = jnp.where(lane == 0, 1.0, 0.0).astype(_F32)

        def orow(r, carry):
            ones[r, pl.ds(0, 16)] = one_row
            return carry

        lax.fori_loop(0, _CH, orow, 0)
        plsc.subcore_barrier()

        def chunk(c, carry):
            pltpu.async_copy(ones, acc.at[idx_all.at[c]], sem, add=True)
            return carry

        lax.fori_loop(0, _NCH, chunk, 0)

        def drain(c, carry):
            pltpu.make_async_copy(ones, acc.at[idx_all.at[c]], sem).wait()
            return carry

        lax.fori_loop(0, _NCH, drain, 0)
        plsc.subcore_barrier()

        pltpu.sync_copy(acc.at[pl.ds(sid * _RPT, _RPT)], tbuf)
        pltpu.sync_copy(tbuf, out_hbm.at[cid, pl.ds(sid * _RPT, _RPT)])

    return count_dst


def _gather_combine(a, b, dst2, src2):
    return _sc_gather_combine()(a, b, dst2, src2)


def _scatter_msg(z, dst2, ac):
    return _sc_scatter_msg()(z, dst2, ac)


def _count_dst(dst2):
    return _sc_count_dst()(dst2)


def _lin_stats(e, w, b, g=None):
    n4, dp = e.shape

    def body(*refs):
        if g is None:
            e_ref, w_ref, b_ref, z_ref, s_ref = refs
            z = jnp.dot(e_ref[...], w_ref[...],
                        preferred_element_type=_F32) + b_ref[...]
        else:
            g_ref, e_ref, w_ref, b_ref, z_ref, s_ref = refs
            z = g_ref[...] + jnp.dot(e_ref[...], w_ref[...],
                                     preferred_element_type=_F32) + b_ref[...]
        z_ref[...] = z

        @pl.when(pl.program_id(0) == 0)
        def _init():
            s_ref[...] = jnp.zeros_like(s_ref)

        s_ref[...] += jnp.stack([jnp.sum(z, 0), jnp.sum(z * z, 0)])

    in_specs = [
        pl.BlockSpec((_BR4, dp), lambda i: (i, 0)),
        pl.BlockSpec((dp, 128), lambda i: (0, 0)),
        pl.BlockSpec((1, 128), lambda i: (0, 0)),
    ]
    args = [e, w, b]
    if g is not None:
        in_specs.insert(0, pl.BlockSpec((_BR4, 128), lambda i: (i, 0)))
        args.insert(0, g)
    return pl.pallas_call(
        body,
        grid=(n4 // _BR4,),
        in_specs=in_specs,
        out_specs=[pl.BlockSpec((_BR4, 128), lambda i: (i, 0)),
                   pl.BlockSpec((2, 128), lambda i: (0, 0))],
        out_shape=[jax.ShapeDtypeStruct((n4, 128), _F32),
                   jax.ShapeDtypeStruct((2, 128), _F32)],
    )(*args)


def _act_lin(z1, vec, w2, with_stats):
    n4 = z1.shape[0]
    dout = w2.shape[1]

    def body(z_ref, v_ref, w_ref, *orefs):
        t = z_ref[...] * v_ref[0:1, :] + v_ref[1:2, :]
        t = jnp.where(t >= 0.0, t, v_ref[2:3, :] * t)
        z2 = jnp.dot(t, w_ref[...],
                     preferred_element_type=_F32) + v_ref[3:4, :dout]
        orefs[0][...] = z2
        if with_stats:
            @pl.when(pl.program_id(0) == 0)
            def _init():
                orefs[1][...] = jnp.zeros_like(orefs[1])
            orefs[1][...] += jnp.stack([jnp.sum(z2, 0), jnp.sum(z2 * z2, 0)])

    out_specs = [pl.BlockSpec((_BR4, dout), lambda i: (i, 0))]
    out_shape = [jax.ShapeDtypeStruct((n4, dout), _F32)]
    if with_stats:
        out_specs.append(pl.BlockSpec((2, dout), lambda i: (0, 0)))
        out_shape.append(jax.ShapeDtypeStruct((2, dout), _F32))
    res = pl.pallas_call(
        body,
        grid=(n4 // _BR4,),
        in_specs=[pl.BlockSpec((_BR4, 128), lambda i: (i, 0)),
                  pl.BlockSpec((4, 128), lambda i: (0, 0)),
                  pl.BlockSpec((128, dout), lambda i: (0, 0))],
        out_specs=out_specs,
        out_shape=out_shape,
    )(z1, vec, w2)
    return res if with_stats else res[0]


def _act_residual(z, vec, base):
    n4 = z.shape[0]

    def body(*refs):
        if base is None:
            z_ref, v_ref, o_ref = refs
        else:
            z_ref, v_ref, b_ref, o_ref = refs
        t = z_ref[...] * v_ref[0:1, :] + v_ref[1:2, :]
        t = jnp.where(t >= 0.0, t, v_ref[2:3, :] * t)
        if base is not None:
            t = b_ref[...] + t
        o_ref[...] = t

    in_specs = [pl.BlockSpec((_BR4, 128), lambda i: (i, 0)),
                pl.BlockSpec((3, 128), lambda i: (0, 0))]
    args = [z, vec]
    if base is not None:
        in_specs.append(pl.BlockSpec((_BR4, 128), lambda i: (i, 0)))
        args.append(base)
    return pl.pallas_call(
        body,
        grid=(n4 // _BR4,),
        in_specs=in_specs,
        out_specs=pl.BlockSpec((_BR4, 128), lambda i: (i, 0)),
        out_shape=jax.ShapeDtypeStruct((n4, 128), _F32),
    )(*args)


def _mlp2_block(x, w1, vec, w2):
    n, din = x.shape
    h = w1.shape[1]

    def bn_act(z, gm, bt, al):
        m = jnp.mean(z, 0, keepdims=True)
        v = jnp.mean(z * z, 0, keepdims=True) - m * m
        t = (z - m) * lax.rsqrt(v + _EPS) * gm + bt
        return jnp.where(t >= 0.0, t, al * t)

    def body(x_ref, w1_ref, v_ref, w2_ref, o_ref):
        z1 = jnp.dot(x_ref[...], w1_ref[...],
                     preferred_element_type=_F32) + v_ref[0:1, :]
        t = bn_act(z1, v_ref[1:2, :], v_ref[2:3, :], v_ref[3:4, :])
        z2 = jnp.dot(t, w2_ref[...],
                     preferred_element_type=_F32) + v_ref[4:5, :]
        o_ref[...] = bn_act(z2, v_ref[5:6, :], v_ref[6:7, :], v_ref[7:8, :])

    return pl.pallas_call(
        body,
        in_specs=[pl.BlockSpec((n, din), lambda: (0, 0)),
                  pl.BlockSpec((din, h), lambda: (0, 0)),
                  pl.BlockSpec((8, h), lambda: (0, 0)),
                  pl.BlockSpec((h, h), lambda: (0, 0))],
        out_specs=pl.BlockSpec((n, h), lambda: (0, 0)),
        out_shape=jax.ShapeDtypeStruct((n, h), _F32),
    )(x, w1, vec, w2)


def _node_update(node, accp, cntp, w1a, w1b, vec, w2, wproj, signs):
    n, h = node.shape
    pw = wproj.shape[1]

    def bn_act(z, gm, bt, al):
        m = jnp.mean(z, 0, keepdims=True)
        v = jnp.mean(z * z, 0, keepdims=True) - m * m
        t = (z - m) * lax.rsqrt(v + _EPS) * gm + bt
        return jnp.where(t >= 0.0, t, al * t)

    def body(nd_ref, ac_ref, ct_ref, wa_ref, wb_ref, v_ref, w2_ref,
             wp_ref, sg_ref, on_ref, op_ref):
        cnt = ct_ref[0] + ct_ref[1]
        inv = 1.0 / jnp.maximum(cnt[:, 0:1], 1.0)
        agg = (ac_ref[0] + ac_ref[1]) * inv
        nd = nd_ref[...]
        z1 = (jnp.dot(nd, wa_ref[...], preferred_element_type=_F32)
              + jnp.dot(agg, wb_ref[...], preferred_element_type=_F32)
              + v_ref[0:1, :])
        t = bn_act(z1, v_ref[1:2, :], v_ref[2:3, :], v_ref[3:4, :])
        z2 = jnp.dot(t, w2_ref[...],
                     preferred_element_type=_F32) + v_ref[4:5, :]
        nd_new = nd + bn_act(z2, v_ref[5:6, :], v_ref[6:7, :], v_ref[7:8, :])
        on_ref[...] = nd_new
        op_ref[...] = jnp.dot(nd_new, wp_ref[...],
                              preferred_element_type=_F32) * sg_ref[...]

    return pl.pallas_call(
        body,
        in_specs=[pl.BlockSpec((n, h), lambda: (0, 0)),
                  pl.BlockSpec((2, n, h), lambda: (0, 0, 0)),
                  pl.BlockSpec((2, n, 16), lambda: (0, 0, 0)),
                  pl.BlockSpec((h, h), lambda: (0, 0)),
                  pl.BlockSpec((h, h), lambda: (0, 0)),
                  pl.BlockSpec((8, h), lambda: (0, 0)),
                  pl.BlockSpec((h, h), lambda: (0, 0)),
                  pl.BlockSpec((h, pw), lambda: (0, 0)),
                  pl.BlockSpec((1, pw), lambda: (0, 0))],
        out_specs=[pl.BlockSpec((n, h), lambda: (0, 0)),
                   pl.BlockSpec((n, pw), lambda: (0, 0))],
        out_shape=[jax.ShapeDtypeStruct((n, h), _F32),
                   jax.ShapeDtypeStruct((n, pw), _F32)],
    )(node, accp, cntp, w1a, w1b, vec, w2, wproj, signs)


def _proj_call(node, wproj):
    n, h = node.shape
    pw = wproj.shape[1]

    def body(nd_ref, wp_ref, o_ref):
        o_ref[...] = jnp.dot(nd_ref[...], wp_ref[...],
                             preferred_element_type=_F32)

    return pl.pallas_call(
        body,
        in_specs=[pl.BlockSpec((n, h), lambda: (0, 0)),
                  pl.BlockSpec((h, pw), lambda: (0, 0))],
        out_specs=pl.BlockSpec((n, pw), lambda: (0, 0)),
        out_shape=jax.ShapeDtypeStruct((n, pw), _F32),
    )(node, wproj)


def _bd4(w):
    return block_diag(w, w, w, w)


def _t4(v):
    return jnp.tile(v, 4)


def _fold_bn(stats, n, bn, alpha):
    s = stats.reshape(2, 4, _H).sum(axis=1)
    m = s[0] / n
    v = s[1] / n - m * m
    a = bn["gamma"] * lax.rsqrt(v + _EPS)
    c = bn["beta"] - m * a
    return jnp.stack([a, c, jnp.full((_H,), alpha, _F32)])


def _vec4(ac3, b2, dout):
    row_b = jnp.zeros((128,), _F32).at[: 4 * dout].set(jnp.tile(b2, 4))
    return jnp.concatenate([jnp.tile(ac3, (1, 4)), row_b[None]], axis=0)


def _mlp_vec(p):
    return jnp.stack([
        p["lin1"]["b"], p["bn1"]["gamma"], p["bn1"]["beta"],
        jnp.full((_H,), p["pr1"]["alpha"], _F32),
        p["lin2"]["b"], p["bn2"]["gamma"], p["bn2"]["beta"],
        jnp.full((_H,), p["pr2"]["alpha"], _F32),
    ])


def kernel(x, edge_index, edge_attr, params):
    src = edge_index[0]
    dst = edge_index[1]
    dst2 = dst.reshape(_E // _CH, _CH)
    src2 = src.reshape(_E // _CH, _CH)
    layers = params["layers"]
    fin = params["final"]

    pe = params["node_emb"]
    node = _mlp2_block(x, pe["lin1"]["w"], _mlp_vec(pe), pe["lin2"]["w"])

    wm1 = layers[0]["msg"]["lin1"]["w"]
    proj = _proj_call(node, jnp.concatenate([wm1[:_H], wm1[_H:2 * _H]], 1))

    pg = params["edge_emb"]
    ea4 = edge_attr.reshape(_E4, 64)
    z1, s1 = _lin_stats(ea4, _bd4(pg["lin1"]["w"]),
                        _t4(pg["lin1"]["b"])[None])
    v1 = _fold_bn(s1, _E, pg["bn1"], pg["pr1"]["alpha"])
    z2, s2 = _act_lin(z1, _vec4(v1, pg["lin2"]["b"], _H),
                      _bd4(pg["lin2"]["w"]), True)
    v2 = _fold_bn(s2, _E, pg["bn2"], pg["pr2"]["alpha"])
    edge = _act_residual(z2, jnp.tile(v2, (1, 4)), None)

    cntp = _count_dst(dst2)

    for li, lp in enumerate(layers):
        mp, np_, ep = lp["msg"], lp["node"], lp["edge"]

        g1 = _gather_combine(proj[:, :_H], proj[:, _H:2 * _H], dst2, src2)
        wm = mp["lin1"]["w"]
        z1, s1 = _lin_stats(edge, _bd4(wm[2 * _H:]),
                            _t4(mp["lin1"]["b"])[None], g=g1)
        v1 = _fold_bn(s1, _E, mp["bn1"], mp["pr1"]["alpha"])
        z2, s2 = _act_lin(z1, _vec4(v1, mp["lin2"]["b"], _H),
                          _bd4(mp["lin2"]["w"]), True)
        ac = _fold_bn(s2, _E, mp["bn2"], mp["pr2"]["alpha"])
        accp = _scatter_msg(z2, dst2, ac)

        we = ep["lin1"]["w"][_H:]
        if li + 1 < len(layers):
            wn = layers[li + 1]["msg"]["lin1"]["w"]
            wp = jnp.concatenate([we, we, wn[:_H], wn[_H:2 * _H]], 1)
            signs = jnp.concatenate([
                jnp.full((1, _H), -1.0, _F32), jnp.full((1, _H), 1.0, _F32),
                jnp.full((1, 2 * _H), 1.0, _F32)], 1)
        else:
            wf = fin["lin1"]["w"][:_H]
            wp = jnp.concatenate([we, we, wf, wf], 1)
            signs = jnp.concatenate([
                jnp.full((1, _H), -1.0, _F32), jnp.full((1, _H), 1.0, _F32),
                jnp.full((1, _H), 1.0, _F32), jnp.full((1, _H), -1.0, _F32)],
                1)
        wn1 = np_["lin1"]["w"]
        node, proj4 = _node_update(node, accp, cntp, wn1[:_H], wn1[_H:],
                                   _mlp_vec(np_), np_["lin2"]["w"], wp, signs)

        d = _gather_combine(proj4[:, :_H], proj4[:, _H:2 * _H], dst2, src2)
        z1, s1 = _lin_stats(edge, _bd4(ep["lin1"]["w"][:_H]),
                            _t4(ep["lin1"]["b"])[None], g=d)
        v1 = _fold_bn(s1, _E, ep["bn1"], ep["pr1"]["alpha"])
        z2, s2 = _act_lin(z1, _vec4(v1, ep["lin2"]["b"], _H),
                          _bd4(ep["lin2"]["w"]), True)
        v2 = _fold_bn(s2, _E, ep["bn2"], ep["pr2"]["alpha"])
        edge = _act_residual(z2, jnp.tile(v2, (1, 4)), edge)
        proj = proj4[:, 2 * _H:]

    gf = _gather_combine(proj[:, :_H], proj[:, _H:], dst2, src2)
    z1, s1 = _lin_stats(edge, _bd4(fin["lin1"]["w"][_H:]),
                        _t4(fin["lin1"]["b"])[None], g=gf)
    v1 = _fold_bn(s1, _E, fin["bn1"], fin["pr1"]["alpha"])
    out4 = _act_lin(z1, _vec4(v1, fin["lin2"]["b"], 3),
                    _bd4(fin["lin2"]["w"]), False)
    return out4.reshape(_E, 3)


# speedup vs baseline: 3.7816x; 1.0230x over previous
import functools

import jax
import jax.numpy as jnp
from jax import lax
from jax.experimental import pallas as pl
from jax.experimental.pallas import tpu as pltpu
from jax.experimental.pallas import tpu_sc as plsc
from jax.scipy.linalg import block_diag

_N = 10000
_E = 320000
_H = 32
_EPS = 1e-5

_NC = 2
_NS = 16
_NW = _NC * _NS
_EPW = _E // _NW
_CH = 80
_CH4 = _CH // 4
_NCH = _EPW // _CH
_RPT = _N // _NS

_E4 = _E // 4
_BR4 = 4000
_F32 = jnp.float32


@functools.cache
def _sc_mesh():
    return plsc.VectorSubcoreMesh(core_axis_name="c", subcore_axis_name="s",
                                  num_cores=_NC, num_subcores=_NS)


@functools.cache
def _sc_gather_combine():
    @functools.partial(
        pl.kernel,
        out_type=jax.ShapeDtypeStruct((_E4, 128), _F32),
        mesh=_sc_mesh(),
        compiler_params=pltpu.CompilerParams(use_tc_tiling_on_sc=False),
        scratch_types=[
            pltpu.VMEM((_NCH, _CH), jnp.int32),
            pltpu.VMEM((_NCH, _CH), jnp.int32),
            pltpu.VMEM((_CH, _H), _F32),
            pltpu.VMEM((_CH, _H), _F32),
            pltpu.VMEM((_CH, _H), _F32),
            pltpu.VMEM((_CH, _H), _F32),
            pltpu.VMEM((_CH4, 128), _F32),
            pltpu.VMEM((_CH4, 128), _F32),
            pltpu.SemaphoreType.DMA,
            pltpu.SemaphoreType.DMA,
            pltpu.SemaphoreType.DMA,
            pltpu.SemaphoreType.DMA,
        ],
    )
    def gather_combine(a_hbm, b_hbm, dst2_hbm, src2_hbm, out_hbm,
                       idx_a, idx_b, ra0, rb0, ra1, rb1, sb0, sb1,
                       sem0, sem1, semw0, semw1):
        wid = lax.axis_index("s") * _NC + lax.axis_index("c")
        wrow = wid * _NCH
        wrow4 = wid * (_EPW // 4)
        pltpu.sync_copy(dst2_hbm.at[pl.ds(wrow, _NCH)], idx_a)
        pltpu.sync_copy(src2_hbm.at[pl.ds(wrow, _NCH)], idx_b)

        def fire(c, ra_, rb_, sem_):
            pltpu.async_copy(a_hbm.at[idx_a.at[c]], ra_, sem_)
            pltpu.async_copy(b_hbm.at[idx_b.at[c]], rb_, sem_)

        def wait_wb(c, sb_, semw_):
            pltpu.make_async_copy(
                sb_, out_hbm.at[pl.ds(wrow4 + c * _CH4, _CH4)], semw_).wait()

        def drain_process(c, ra_, rb_, sb_, sem_, semw_):
            pltpu.make_async_copy(a_hbm.at[idx_a.at[c]], ra_, sem_).wait()
            pltpu.make_async_copy(b_hbm.at[idx_b.at[c]], rb_, sem_).wait()

            @pl.when(c >= 2)
            def _():
                wait_wb(c - 2, sb_, semw_)

            def addq(q, c2):
                r0 = q * 4
                for k in range(4):
                    for h in range(2):
                        sl = pl.ds(h * 16, 16)
                        sb_[q, pl.ds(k * _H + h * 16, 16)] = (
                            ra_[r0 + k, sl] + rb_[r0 + k, sl])
                return c2

            lax.fori_loop(0, _CH4, addq, 0, unroll=2)
            pltpu.async_copy(sb_, out_hbm.at[pl.ds(wrow4 + c * _CH4, _CH4)],
                             semw_)

        fire(0, ra0, rb0, sem0)

        def pair(p, carry):
            c0 = 2 * p
            fire(c0 + 1, ra1, rb1, sem1)
            drain_process(c0, ra0, rb0, sb0, sem0, semw0)

            @pl.when(c0 + 2 < _NCH)
            def _():
                fire(c0 + 2, ra0, rb0, sem0)

            drain_process(c0 + 1, ra1, rb1, sb1, sem1, semw1)
            return carry

        lax.fori_loop(0, _NCH // 2, pair, 0)
        drain_process(_NCH - 1, ra0, rb0, sb0, sem0, semw0)
        wait_wb(_NCH - 1, sb0, semw0)
        wait_wb(_NCH - 2, sb1, semw1)

    return gather_combine


@functools.cache
def _sc_scatter_msg():
    @functools.partial(
        pl.kernel,
        out_type=jax.ShapeDtypeStruct((_NC, _N, _H), _F32),
        mesh=_sc_mesh(),
        compiler_params=pltpu.CompilerParams(use_tc_tiling_on_sc=False),
        scratch_types=[
            pltpu.VMEM((_NCH, _CH), jnp.int32),
            pltpu.VMEM((_CH4, 128), _F32),
            pltpu.VMEM((_CH4, 128), _F32),
            pltpu.VMEM((_CH, _H), _F32),
            pltpu.VMEM((_CH, _H), _F32),
            pltpu.VMEM((_RPT, _H), _F32),
            pltpu.VMEM((3, _H), _F32),
            pltpu.VMEM_SHARED((_N, _H), _F32),
            pltpu.SemaphoreType.DMA,
            pltpu.SemaphoreType.DMA,
            pltpu.SemaphoreType.DMA,
            pltpu.SemaphoreType.DMA,
        ],
    )
    def scatter_msg(z_hbm, dst2_hbm, ac_hbm, out_hbm,
                    idx_all, zp0, zp1, zs0, zs1, tbuf, acv, acc,
                    semz0, semz1, sems0, sems1):
        cid = lax.axis_index("c")
        sid = lax.axis_index("s")
        wid = sid * _NC + cid
        wrow = wid * _NCH
        wrow4 = wid * (_EPW // 4)

        def zrow(r, carry):
            for h in range(2):
                tbuf[r, pl.ds(h * 16, 16)] = jnp.zeros((16,), _F32)
            return carry

        lax.fori_loop(0, _RPT, zrow, 0)
        pltpu.sync_copy(tbuf, acc.at[pl.ds(sid * _RPT, _RPT)])
        pltpu.sync_copy(dst2_hbm.at[pl.ds(wrow, _NCH)], idx_all)
        plsc.subcore_barrier()

        pltpu.sync_copy(ac_hbm, acv)
        a_lo = acv[0, pl.ds(0, 16)]
        a_hi = acv[0, pl.ds(16, 16)]
        c_lo = acv[1, pl.ds(0, 16)]
        c_hi = acv[1, pl.ds(16, 16)]
        p_lo = acv[2, pl.ds(0, 16)]
        p_hi = acv[2, pl.ds(16, 16)]

        def fire_z(c, zp_, semz_):
            pltpu.async_copy(z_hbm.at[pl.ds(wrow4 + c * _CH4, _CH4)], zp_,
                             semz_)

        def process(c, zp_, zs_, semz_, sems_):
            pltpu.make_async_copy(
                z_hbm.at[pl.ds(wrow4 + c * _CH4, _CH4)], zp_, semz_).wait()

            def pq(q, c2):
                r0 = q * 4
                for k in range(4):
                    for h, (av, cv, pv) in enumerate(((a_lo, c_lo, p_lo),
                                                      (a_hi, c_hi, p_hi))):
                        v = zp_[q, pl.ds(k * _H + h * 16, 16)] * av + cv
                        zs_[r0 + k, pl.ds(h * 16, 16)] = jnp.where(
                            v >= 0.0, v, pv * v)
                return c2

            lax.fori_loop(0, _CH4, pq, 0, unroll=2)
            pltpu.async_copy(zs_, acc.at[idx_all.at[c]], sems_, add=True)

        def wait_scat(c, zs_, sems_):
            pltpu.make_async_copy(zs_, acc.at[idx_all.at[c]], sems_).wait()

        fire_z(0, zp0, semz0)
        fire_z(1, zp1, semz1)

        def pair(p, carry):
            c0 = 2 * p
            process(c0, zp0, zs0, semz0, sems0)

            @pl.when(c0 + 2 < _NCH)
            def _():
                wait_scat(c0, zs0, sems0)
                fire_z(c0 + 2, zp0, semz0)

            process(c0 + 1, zp1, zs1, semz1, sems1)

            @pl.when(c0 + 3 < _NCH)
            def _():
                wait_scat(c0 + 1, zs1, sems1)
                fire_z(c0 + 3, zp1, semz1)

            return carry

        lax.fori_loop(0, _NCH // 2, pair, 0)
        process(_NCH - 1, zp0, zs0, semz0, sems0)
        wait_scat(_NCH - 1, zs0, sems0)
        wait_scat(_NCH - 2, zs1, sems1)
        plsc.subcore_barrier()

        pltpu.sync_copy(acc.at[pl.ds(sid * _RPT, _RPT)], tbuf)
        pltpu.sync_copy(tbuf, out_hbm.at[cid, pl.ds(sid * _RPT, _RPT)])

    return scatter_msg


@functools.cache
def _sc_count_dst():
    @functools.partial(
        pl.kernel,
        out_type=jax.ShapeDtypeStruct((_NC, _N, 16), _F32),
        mesh=_sc_mesh(),
        compiler_params=pltpu.CompilerParams(use_tc_tiling_on_sc=False),
        scratch_types=[
            pltpu.VMEM((_NCH, _CH), jnp.int32),
            pltpu.VMEM((_CH, 16), _F32),
            pltpu.VMEM((_RPT, 16), _F32),
            pltpu.VMEM_SHARED((_N, 16), _F32),
            pltpu.SemaphoreType.DMA,
        ],
    )
    def count_dst(dst2_hbm, out_hbm, idx_all, ones, tbuf, acc, sem):
        cid = lax.axis_index("c")
        sid = lax.axis_index("s")
        wid = sid * _NC + cid
        wrow = wid * _NCH

        def zrow(r, carry):
            tbuf[r, pl.ds(0, 16)] = jnp.zeros((16,), _F32)
            return carry

        lax.fori_loop(0, _RPT, zrow, 0)
        pltpu.sync_copy(tbuf, acc.at[pl.ds(sid * _RPT, _RPT)])
        pltpu.sync_copy(dst2_hbm.at[pl.ds(wrow, _NCH)], idx_all)

        lane = lax.broadcasted_iota(jnp.int32, (16,), 0)
        one_row = jnp.where(lane == 0, 1.0, 0.0).astype(_F32)

        def orow(r, carry):
            ones[r, pl.ds(0, 16)] = one_row
            return carry

        lax.fori_loop(0, _CH, orow, 0)
        plsc.subcore_barrier()

        def chunk(c, carry):
            pltpu.async_copy(ones, acc.at[idx_all.at[c]], sem, add=True)
            return carry

        lax.fori_loop(0, _NCH, chunk, 0)

        def drain(c, carry):
            pltpu.make_async_copy(ones, acc.at[idx_all.at[c]], sem).wait()
            return carry

        lax.fori_loop(0, _NCH, drain, 0)
        plsc.subcore_barrier()

        pltpu.sync_copy(acc.at[pl.ds(sid * _RPT, _RPT)], tbuf)
        pltpu.sync_copy(tbuf, out_hbm.at[cid, pl.ds(sid * _RPT, _RPT)])

    return count_dst


def _gather_combine(a, b, dst2, src2):
    return _sc_gather_combine()(a, b, dst2, src2)


def _scatter_msg(z, dst2, ac):
    return _sc_scatter_msg()(z, dst2, ac)


def _count_dst(dst2):
    return _sc_count_dst()(dst2)


def _lin_stats(e, w, b, g=None):
    n4, dp = e.shape

    def body(*refs):
        if g is None:
            e_ref, w_ref, b_ref, z_ref, s_ref = refs
            z = jnp.dot(e_ref[...], w_ref[...],
                        preferred_element_type=_F32) + b_ref[...]
        else:
            g_ref, e_ref, w_ref, b_ref, z_ref, s_ref = refs
            z = g_ref[...] + jnp.dot(e_ref[...], w_ref[...],
                                     preferred_element_type=_F32) + b_ref[...]
        z_ref[...] = z

        @pl.when(pl.program_id(0) == 0)
        def _init():
            s_ref[...] = jnp.zeros_like(s_ref)

        s_ref[...] += jnp.stack([jnp.sum(z, 0), jnp.sum(z * z, 0)])

    in_specs = [
        pl.BlockSpec((_BR4, dp), lambda i: (i, 0)),
        pl.BlockSpec((dp, 128), lambda i: (0, 0)),
        pl.BlockSpec((1, 128), lambda i: (0, 0)),
    ]
    args = [e, w, b]
    if g is not None:
        in_specs.insert(0, pl.BlockSpec((_BR4, 128), lambda i: (i, 0)))
        args.insert(0, g)
    return pl.pallas_call(
        body,
        grid=(n4 // _BR4,),
        in_specs=in_specs,
        out_specs=[pl.BlockSpec((_BR4, 128), lambda i: (i, 0)),
                   pl.BlockSpec((2, 128), lambda i: (0, 0))],
        out_shape=[jax.ShapeDtypeStruct((n4, 128), _F32),
                   jax.ShapeDtypeStruct((2, 128), _F32)],
    )(*args)


def _act_lin(z1, vec, w2, with_stats):
    n4 = z1.shape[0]
    dout = w2.shape[1]

    def body(z_ref, v_ref, w_ref, *orefs):
        t = z_ref[...] * v_ref[0:1, :] + v_ref[1:2, :]
        t = jnp.where(t >= 0.0, t, v_ref[2:3, :] * t)
        z2 = jnp.dot(t, w_ref[...],
                     preferred_element_type=_F32) + v_ref[3:4, :dout]
        orefs[0][...] = z2
        if with_stats:
            @pl.when(pl.program_id(0) == 0)
            def _init():
                orefs[1][...] = jnp.zeros_like(orefs[1])
            orefs[1][...] += jnp.stack([jnp.sum(z2, 0), jnp.sum(z2 * z2, 0)])

    out_specs = [pl.BlockSpec((_BR4, dout), lambda i: (i, 0))]
    out_shape = [jax.ShapeDtypeStruct((n4, dout), _F32)]
    if with_stats:
        out_specs.append(pl.BlockSpec((2, dout), lambda i: (0, 0)))
        out_shape.append(jax.ShapeDtypeStruct((2, dout), _F32))
    res = pl.pallas_call(
        body,
        grid=(n4 // _BR4,),
        in_specs=[pl.BlockSpec((_BR4, 128), lambda i: (i, 0)),
                  pl.BlockSpec((4, 128), lambda i: (0, 0)),
                  pl.BlockSpec((128, dout), lambda i: (0, 0))],
        out_specs=out_specs,
        out_shape=out_shape,
    )(z1, vec, w2)
    return res if with_stats else res[0]


def _act_residual(z, vec, base):
    n4 = z.shape[0]

    def body(*refs):
        if base is None:
            z_ref, v_ref, o_ref = refs
        else:
            z_ref, v_ref, b_ref, o_ref = refs
        t = z_ref[...] * v_ref[0:1, :] + v_ref[1:2, :]
        t = jnp.where(t >= 0.0, t, v_ref[2:3, :] * t)
        if base is not None:
            t = b_ref[...] + t
        o_ref[...] = t

    in_specs = [pl.BlockSpec((_BR4, 128), lambda i: (i, 0)),
                pl.BlockSpec((3, 128), lambda i: (0, 0))]
    args = [z, vec]
    if base is not None:
        in_specs.append(pl.BlockSpec((_BR4, 128), lambda i: (i, 0)))
        args.append(base)
    return pl.pallas_call(
        body,
        grid=(n4 // _BR4,),
        in_specs=in_specs,
        out_specs=pl.BlockSpec((_BR4, 128), lambda i: (i, 0)),
        out_shape=jax.ShapeDtypeStruct((n4, 128), _F32),
    )(*args)


def _mlp2_block(x, w1, vec, w2):
    n, din = x.shape
    h = w1.shape[1]

    def bn_act(z, gm, bt, al):
        m = jnp.mean(z, 0, keepdims=True)
        v = jnp.mean(z * z, 0, keepdims=True) - m * m
        t = (z - m) * lax.rsqrt(v + _EPS) * gm + bt
        return jnp.where(t >= 0.0, t, al * t)

    def body(x_ref, w1_ref, v_ref, w2_ref, o_ref):
        z1 = jnp.dot(x_ref[...], w1_ref[...],
                     preferred_element_type=_F32) + v_ref[0:1, :]
        t = bn_act(z1, v_ref[1:2, :], v_ref[2:3, :], v_ref[3:4, :])
        z2 = jnp.dot(t, w2_ref[...],
                     preferred_element_type=_F32) + v_ref[4:5, :]
        o_ref[...] = bn_act(z2, v_ref[5:6, :], v_ref[6:7, :], v_ref[7:8, :])

    return pl.pallas_call(
        body,
        in_specs=[pl.BlockSpec((n, din), lambda: (0, 0)),
                  pl.BlockSpec((din, h), lambda: (0, 0)),
                  pl.BlockSpec((8, h), lambda: (0, 0)),
                  pl.BlockSpec((h, h), lambda: (0, 0))],
        out_specs=pl.BlockSpec((n, h), lambda: (0, 0)),
        out_shape=jax.ShapeDtypeStruct((n, h), _F32),
    )(x, w1, vec, w2)


def _node_update(node, accp, cntp, w1a, w1b, vec, w2, wproj, signs):
    n, h = node.shape
    pw = wproj.shape[1]

    def bn_act(z, gm, bt, al):
        m = jnp.mean(z, 0, keepdims=True)
        v = jnp.mean(z * z, 0, keepdims=True) - m * m
        t = (z - m) * lax.rsqrt(v + _EPS) * gm + bt
        return jnp.where(t >= 0.0, t, al * t)

    def body(nd_ref, ac_ref, ct_ref, wa_ref, wb_ref, v_ref, w2_ref,
             wp_ref, sg_ref, on_ref, op_ref):
        cnt = ct_ref[0] + ct_ref[1]
        inv = 1.0 / jnp.maximum(cnt[:, 0:1], 1.0)
        agg = (ac_ref[0] + ac_ref[1]) * inv
        nd = nd_ref[...]
        z1 = (jnp.dot(nd, wa_ref[...], preferred_element_type=_F32)
              + jnp.dot(agg, wb_ref[...], preferred_element_type=_F32)
              + v_ref[0:1, :])
        t = bn_act(z1, v_ref[1:2, :], v_ref[2:3, :], v_ref[3:4, :])
        z2 = jnp.dot(t, w2_ref[...],
                     preferred_element_type=_F32) + v_ref[4:5, :]
        nd_new = nd + bn_act(z2, v_ref[5:6, :], v_ref[6:7, :], v_ref[7:8, :])
        on_ref[...] = nd_new
        op_ref[...] = jnp.dot(nd_new, wp_ref[...],
                              preferred_element_type=_F32) * sg_ref[...]

    return pl.pallas_call(
        body,
        in_specs=[pl.BlockSpec((n, h), lambda: (0, 0)),
                  pl.BlockSpec((2, n, h), lambda: (0, 0, 0)),
                  pl.BlockSpec((2, n, 16), lambda: (0, 0, 0)),
                  pl.BlockSpec((h, h), lambda: (0, 0)),
                  pl.BlockSpec((h, h), lambda: (0, 0)),
                  pl.BlockSpec((8, h), lambda: (0, 0)),
                  pl.BlockSpec((h, h), lambda: (0, 0)),
                  pl.BlockSpec((h, pw), lambda: (0, 0)),
                  pl.BlockSpec((1, pw), lambda: (0, 0))],
        out_specs=[pl.BlockSpec((n, h), lambda: (0, 0)),
                   pl.BlockSpec((n, pw), lambda: (0, 0))],
        out_shape=[jax.ShapeDtypeStruct((n, h), _F32),
                   jax.ShapeDtypeStruct((n, pw), _F32)],
    )(node, accp, cntp, w1a, w1b, vec, w2, wproj, signs)


def _proj_call(node, wproj):
    n, h = node.shape
    pw = wproj.shape[1]

    def body(nd_ref, wp_ref, o_ref):
        o_ref[...] = jnp.dot(nd_ref[...], wp_ref[...],
                             preferred_element_type=_F32)

    return pl.pallas_call(
        body,
        in_specs=[pl.BlockSpec((n, h), lambda: (0, 0)),
                  pl.BlockSpec((h, pw), lambda: (0, 0))],
        out_specs=pl.BlockSpec((n, pw), lambda: (0, 0)),
        out_shape=jax.ShapeDtypeStruct((n, pw), _F32),
    )(node, wproj)


def _bd4(w):
    return block_diag(w, w, w, w)


def _t4(v):
    return jnp.tile(v, 4)


def _fold_bn(stats, n, bn, alpha):
    s = stats.reshape(2, 4, _H).sum(axis=1)
    m = s[0] / n
    v = s[1] / n - m * m
    a = bn["gamma"] * lax.rsqrt(v + _EPS)
    c = bn["beta"] - m * a
    return jnp.stack([a, c, jnp.full((_H,), alpha, _F32)])


def _vec4(ac3, b2, dout):
    row_b = jnp.zeros((128,), _F32).at[: 4 * dout].set(jnp.tile(b2, 4))
    return jnp.concatenate([jnp.tile(ac3, (1, 4)), row_b[None]], axis=0)


def _mlp_vec(p):
    return jnp.stack([
        p["lin1"]["b"], p["bn1"]["gamma"], p["bn1"]["beta"],
        jnp.full((_H,), p["pr1"]["alpha"], _F32),
        p["lin2"]["b"], p["bn2"]["gamma"], p["bn2"]["beta"],
        jnp.full((_H,), p["pr2"]["alpha"], _F32),
    ])


def kernel(x, edge_index, edge_attr, params):
    src = edge_index[0]
    dst = edge_index[1]
    dst2 = dst.reshape(_E // _CH, _CH)
    src2 = src.reshape(_E // _CH, _CH)
    layers = params["layers"]
    fin = params["final"]

    pe = params["node_emb"]
    node = _mlp2_block(x, pe["lin1"]["w"], _mlp_vec(pe), pe["lin2"]["w"])

    wm1 = layers[0]["msg"]["lin1"]["w"]
    proj = _proj_call(node, jnp.concatenate([wm1[:_H], wm1[_H:2 * _H]], 1))

    pg = params["edge_emb"]
    ea4 = edge_attr.reshape(_E4, 64)
    z1, s1 = _lin_stats(ea4, _bd4(pg["lin1"]["w"]),
                        _t4(pg["lin1"]["b"])[None])
    v1 = _fold_bn(s1, _E, pg["bn1"], pg["pr1"]["alpha"])
    z2, s2 = _act_lin(z1, _vec4(v1, pg["lin2"]["b"], _H),
                      _bd4(pg["lin2"]["w"]), True)
    v2 = _fold_bn(s2, _E, pg["bn2"], pg["pr2"]["alpha"])
    edge = _act_residual(z2, jnp.tile(v2, (1, 4)), None)

    cntp = _count_dst(dst2)

    for li, lp in enumerate(layers):
        mp, np_, ep = lp["msg"], lp["node"], lp["edge"]

        g1 = _gather_combine(proj[:, :_H], proj[:, _H:2 * _H], dst2, src2)
        wm = mp["lin1"]["w"]
        z1, s1 = _lin_stats(edge, _bd4(wm[2 * _H:]),
                            _t4(mp["lin1"]["b"])[None], g=g1)
        v1 = _fold_bn(s1, _E, mp["bn1"], mp["pr1"]["alpha"])
        z2, s2 = _act_lin(z1, _vec4(v1, mp["lin2"]["b"], _H),
                          _bd4(mp["lin2"]["w"]), True)
        ac = _fold_bn(s2, _E, mp["bn2"], mp["pr2"]["alpha"])
        accp = _scatter_msg(z2, dst2, ac)

        we = ep["lin1"]["w"][_H:]
        if li + 1 < len(layers):
            wn = layers[li + 1]["msg"]["lin1"]["w"]
            wp = jnp.concatenate([we, we, wn[:_H], wn[_H:2 * _H]], 1)
            signs = jnp.concatenate([
                jnp.full((1, _H), -1.0, _F32), jnp.full((1, _H), 1.0, _F32),
                jnp.full((1, 2 * _H), 1.0, _F32)], 1)
        else:
            wf = fin["lin1"]["w"][:_H]
            wp = jnp.concatenate([we, we, wf, wf], 1)
            signs = jnp.concatenate([
                jnp.full((1, _H), -1.0, _F32), jnp.full((1, _H), 1.0, _F32),
                jnp.full((1, _H), 1.0, _F32), jnp.full((1, _H), -1.0, _F32)],
                1)
        wn1 = np_["lin1"]["w"]
        node, proj4 = _node_update(node, accp, cntp, wn1[:_H], wn1[_H:],
                                   _mlp_vec(np_), np_["lin2"]["w"], wp, signs)

        d = _gather_combine(proj4[:, :_H], proj4[:, _H:2 * _H], dst2, src2)
        z1, s1 = _lin_stats(edge, _bd4(ep["lin1"]["w"][:_H]),
                            _t4(ep["lin1"]["b"])[None], g=d)
        v1 = _fold_bn(s1, _E, ep["bn1"], ep["pr1"]["alpha"])
        z2, s2 = _act_lin(z1, _vec4(v1, ep["lin2"]["b"], _H),
                          _bd4(ep["lin2"]["w"]), True)
        v2 = _fold_bn(s2, _E, ep["bn2"], ep["pr2"]["alpha"])
        edge = _act_residual(z2, jnp.tile(v2, (1, 4)), edge)
        proj = proj4[:, 2 * _H:]

    gf = _gather_combine(proj[:, :_H], proj[:, _H:], dst2, src2)
    z1, s1 = _lin_stats(edge, _bd4(fin["lin1"]["w"][_H:]),
                        _t4(fin["lin1"]["b"])[None], g=gf)
    v1 = _fold_bn(s1, _E, fin["bn1"], fin["pr1"]["alpha"])
    out4 = _act_lin(z1, _vec4(v1, fin["lin2"]["b"], 3),
                    _bd4(fin["lin2"]["w"]), False)
    return out4.reshape(_E, 3)


# speedup vs baseline: 3.8793x; 1.0258x over previous
import functools

import jax
import jax.numpy as jnp
from jax import lax
from jax.experimental import pallas as pl
from jax.experimental.pallas import tpu as pltpu
from jax.experimental.pallas import tpu_sc as plsc
from jax.scipy.linalg import block_diag

_N = 10000
_E = 320000
_H = 32
_EPS = 1e-5

_NC = 2
_NS = 16
_NW = _NC * _NS
_EPW = _E // _NW
_CH = 80
_CH4 = _CH // 4
_NCH = _EPW // _CH
_RPT = _N // _NS

_E4 = _E // 4
_BR4 = 4000
_F32 = jnp.float32


@functools.cache
def _sc_mesh():
    return plsc.VectorSubcoreMesh(core_axis_name="c", subcore_axis_name="s",
                                  num_cores=_NC, num_subcores=_NS)


@functools.cache
def _sc_gather_combine():
    @functools.partial(
        pl.kernel,
        out_type=jax.ShapeDtypeStruct((_E4, 128), _F32),
        mesh=_sc_mesh(),
        compiler_params=pltpu.CompilerParams(use_tc_tiling_on_sc=False),
        scratch_types=(
            [pltpu.VMEM((_NCH, _CH), jnp.int32)] * 2
            + [pltpu.VMEM((_CH, _H), _F32)] * 8
            + [pltpu.VMEM((_CH4, 128), _F32)] * 4
            + [pltpu.SemaphoreType.DMA] * 8
        ),
    )
    def gather_combine(a_hbm, b_hbm, dst2_hbm, src2_hbm, out_hbm,
                       idx_a, idx_b,
                       ra0, rb0, ra1, rb1, ra2, rb2, ra3, rb3,
                       sb0, sb1, sb2, sb3,
                       sem0, sem1, sem2, sem3,
                       semw0, semw1, semw2, semw3):
        wid = lax.axis_index("s") * _NC + lax.axis_index("c")
        wrow = wid * _NCH
        wrow4 = wid * (_EPW // 4)
        pltpu.sync_copy(dst2_hbm.at[pl.ds(wrow, _NCH)], idx_a)
        pltpu.sync_copy(src2_hbm.at[pl.ds(wrow, _NCH)], idx_b)

        bufs = ((ra0, rb0, sb0, sem0, semw0),
                (ra1, rb1, sb1, sem1, semw1),
                (ra2, rb2, sb2, sem2, semw2),
                (ra3, rb3, sb3, sem3, semw3))

        def fire(c, ra_, rb_, sem_):
            pltpu.async_copy(a_hbm.at[idx_a.at[c]], ra_, sem_)
            pltpu.async_copy(b_hbm.at[idx_b.at[c]], rb_, sem_)

        def wait_wb(c, sb_, semw_):
            pltpu.make_async_copy(
                sb_, out_hbm.at[pl.ds(wrow4 + c * _CH4, _CH4)], semw_).wait()

        def drain_process(c, ra_, rb_, sb_, sem_, semw_):
            pltpu.make_async_copy(a_hbm.at[idx_a.at[c]], ra_, sem_).wait()
            pltpu.make_async_copy(b_hbm.at[idx_b.at[c]], rb_, sem_).wait()

            @pl.when(c >= 4)
            def _():
                wait_wb(c - 4, sb_, semw_)

            def addq(q, c2):
                r0 = q * 4
                for k in range(4):
                    for h in range(2):
                        sl = pl.ds(h * 16, 16)
                        sb_[q, pl.ds(k * _H + h * 16, 16)] = (
                            ra_[r0 + k, sl] + rb_[r0 + k, sl])
                return c2

            lax.fori_loop(0, _CH4, addq, 0, unroll=2)
            pltpu.async_copy(sb_, out_hbm.at[pl.ds(wrow4 + c * _CH4, _CH4)],
                             semw_)

        for k in range(4):
            fire(k, bufs[k][0], bufs[k][1], bufs[k][3])

        def quad(p, carry):
            c0 = 4 * p
            for k in range(4):
                ra_, rb_, sb_, sem_, semw_ = bufs[k]
                c = c0 + k
                drain_process(c, ra_, rb_, sb_, sem_, semw_)

                @pl.when(c + 4 < _NCH)
                def _():
                    fire(c + 4, ra_, rb_, sem_)

            return carry

        lax.fori_loop(0, _NCH // 4, quad, 0)
        drain_process(_NCH - 1, *bufs[0])
        wait_wb(_NCH - 1, bufs[0][2], bufs[0][4])
        for k in range(1, 4):
            wait_wb(_NCH - 5 + k, bufs[k][2], bufs[k][4])

    return gather_combine


@functools.cache
def _sc_scatter_msg():
    @functools.partial(
        pl.kernel,
        out_type=jax.ShapeDtypeStruct((_NC, _N, _H), _F32),
        mesh=_sc_mesh(),
        compiler_params=pltpu.CompilerParams(use_tc_tiling_on_sc=False),
        scratch_types=[
            pltpu.VMEM((_NCH, _CH), jnp.int32),
            pltpu.VMEM((_CH4, 128), _F32),
            pltpu.VMEM((_CH4, 128), _F32),
            pltpu.VMEM((_CH, _H), _F32),
            pltpu.VMEM((_CH, _H), _F32),
            pltpu.VMEM((_RPT, _H), _F32),
            pltpu.VMEM((3, _H), _F32),
            pltpu.VMEM_SHARED((_N, _H), _F32),
            pltpu.SemaphoreType.DMA,
            pltpu.SemaphoreType.DMA,
            pltpu.SemaphoreType.DMA,
            pltpu.SemaphoreType.DMA,
        ],
    )
    def scatter_msg(z_hbm, dst2_hbm, ac_hbm, out_hbm,
                    idx_all, zp0, zp1, zs0, zs1, tbuf, acv, acc,
                    semz0, semz1, sems0, sems1):
        cid = lax.axis_index("c")
        sid = lax.axis_index("s")
        wid = sid * _NC + cid
        wrow = wid * _NCH
        wrow4 = wid * (_EPW // 4)

        def zrow(r, carry):
            for h in range(2):
                tbuf[r, pl.ds(h * 16, 16)] = jnp.zeros((16,), _F32)
            return carry

        lax.fori_loop(0, _RPT, zrow, 0)
        pltpu.sync_copy(tbuf, acc.at[pl.ds(sid * _RPT, _RPT)])
        pltpu.sync_copy(dst2_hbm.at[pl.ds(wrow, _NCH)], idx_all)
        plsc.subcore_barrier()

        pltpu.sync_copy(ac_hbm, acv)
        a_lo = acv[0, pl.ds(0, 16)]
        a_hi = acv[0, pl.ds(16, 16)]
        c_lo = acv[1, pl.ds(0, 16)]
        c_hi = acv[1, pl.ds(16, 16)]
        p_lo = acv[2, pl.ds(0, 16)]
        p_hi = acv[2, pl.ds(16, 16)]

        def fire_z(c, zp_, semz_):
            pltpu.async_copy(z_hbm.at[pl.ds(wrow4 + c * _CH4, _CH4)], zp_,
                             semz_)

        def process(c, zp_, zs_, semz_, sems_):
            pltpu.make_async_copy(
                z_hbm.at[pl.ds(wrow4 + c * _CH4, _CH4)], zp_, semz_).wait()

            def pq(q, c2):
                r0 = q * 4
                for k in range(4):
                    for h, (av, cv, pv) in enumerate(((a_lo, c_lo, p_lo),
                                                      (a_hi, c_hi, p_hi))):
                        v = zp_[q, pl.ds(k * _H + h * 16, 16)] * av + cv
                        zs_[r0 + k, pl.ds(h * 16, 16)] = jnp.where(
                            v >= 0.0, v, pv * v)
                return c2

            lax.fori_loop(0, _CH4, pq, 0, unroll=2)
            pltpu.async_copy(zs_, acc.at[idx_all.at[c]], sems_, add=True)

        def wait_scat(c, zs_, sems_):
            pltpu.make_async_copy(zs_, acc.at[idx_all.at[c]], sems_).wait()

        fire_z(0, zp0, semz0)
        fire_z(1, zp1, semz1)

        def pair(p, carry):
            c0 = 2 * p
            process(c0, zp0, zs0, semz0, sems0)

            @pl.when(c0 + 2 < _NCH)
            def _():
                wait_scat(c0, zs0, sems0)
                fire_z(c0 + 2, zp0, semz0)

            process(c0 + 1, zp1, zs1, semz1, sems1)

            @pl.when(c0 + 3 < _NCH)
            def _():
                wait_scat(c0 + 1, zs1, sems1)
                fire_z(c0 + 3, zp1, semz1)

            return carry

        lax.fori_loop(0, _NCH // 2, pair, 0)
        process(_NCH - 1, zp0, zs0, semz0, sems0)
        wait_scat(_NCH - 1, zs0, sems0)
        wait_scat(_NCH - 2, zs1, sems1)
        plsc.subcore_barrier()

        pltpu.sync_copy(acc.at[pl.ds(sid * _RPT, _RPT)], tbuf)
        pltpu.sync_copy(tbuf, out_hbm.at[cid, pl.ds(sid * _RPT, _RPT)])

    return scatter_msg


@functools.cache
def _sc_count_dst():
    @functools.partial(
        pl.kernel,
        out_type=jax.ShapeDtypeStruct((_NC, _N, 16), _F32),
        mesh=_sc_mesh(),
        compiler_params=pltpu.CompilerParams(use_tc_tiling_on_sc=False),
        scratch_types=[
            pltpu.VMEM((_NCH, _CH), jnp.int32),
            pltpu.VMEM((_CH, 16), _F32),
            pltpu.VMEM((_RPT, 16), _F32),
            pltpu.VMEM_SHARED((_N, 16), _F32),
            pltpu.SemaphoreType.DMA,
        ],
    )
    def count_dst(dst2_hbm, out_hbm, idx_all, ones, tbuf, acc, sem):
        cid = lax.axis_index("c")
        sid = lax.axis_index("s")
        wid = sid * _NC + cid
        wrow = wid * _NCH

        def zrow(r, carry):
            tbuf[r, pl.ds(0, 16)] = jnp.zeros((16,), _F32)
            return carry

        lax.fori_loop(0, _RPT, zrow, 0)
        pltpu.sync_copy(tbuf, acc.at[pl.ds(sid * _RPT, _RPT)])
        pltpu.sync_copy(dst2_hbm.at[pl.ds(wrow, _NCH)], idx_all)

        lane = lax.broadcasted_iota(jnp.int32, (16,), 0)
        one_row = jnp.where(lane == 0, 1.0, 0.0).astype(_F32)

        def orow(r, carry):
            ones[r, pl.ds(0, 16)] = one_row
            return carry

        lax.fori_loop(0, _CH, orow, 0)
        plsc.subcore_barrier()

        def chunk(c, carry):
            pltpu.async_copy(ones, acc.at[idx_all.at[c]], sem, add=True)
            return carry

        lax.fori_loop(0, _NCH, chunk, 0)

        def drain(c, carry):
            pltpu.make_async_copy(ones, acc.at[idx_all.at[c]], sem).wait()
            return carry

        lax.fori_loop(0, _NCH, drain, 0)
        plsc.subcore_barrier()

        pltpu.sync_copy(acc.at[pl.ds(sid * _RPT, _RPT)], tbuf)
        pltpu.sync_copy(tbuf, out_hbm.at[cid, pl.ds(sid * _RPT, _RPT)])

    return count_dst


def _gather_combine(a, b, dst2, src2):
    return _sc_gather_combine()(a, b, dst2, src2)


def _scatter_msg(z, dst2, ac):
    return _sc_scatter_msg()(z, dst2, ac)


def _count_dst(dst2):
    return _sc_count_dst()(dst2)


def _lin_stats(e, w, b, g=None):
    n4, dp = e.shape

    def body(*refs):
        if g is None:
            e_ref, w_ref, b_ref, z_ref, s_ref = refs
            z = jnp.dot(e_ref[...], w_ref[...],
                        preferred_element_type=_F32) + b_ref[...]
        else:
            g_ref, e_ref, w_ref, b_ref, z_ref, s_ref = refs
            z = g_ref[...] + jnp.dot(e_ref[...], w_ref[...],
                                     preferred_element_type=_F32) + b_ref[...]
        z_ref[...] = z

        @pl.when(pl.program_id(0) == 0)
        def _init():
            s_ref[...] = jnp.zeros_like(s_ref)

        s_ref[...] += jnp.stack([jnp.sum(z, 0), jnp.sum(z * z, 0)])

    in_specs = [
        pl.BlockSpec((_BR4, dp), lambda i: (i, 0)),
        pl.BlockSpec((dp, 128), lambda i: (0, 0)),
        pl.BlockSpec((1, 128), lambda i: (0, 0)),
    ]
    args = [e, w, b]
    if g is not None:
        in_specs.insert(0, pl.BlockSpec((_BR4, 128), lambda i: (i, 0)))
        args.insert(0, g)
    return pl.pallas_call(
        body,
        grid=(n4 // _BR4,),
        in_specs=in_specs,
        out_specs=[pl.BlockSpec((_BR4, 128), lambda i: (i, 0)),
                   pl.BlockSpec((2, 128), lambda i: (0, 0))],
        out_shape=[jax.ShapeDtypeStruct((n4, 128), _F32),
                   jax.ShapeDtypeStruct((2, 128), _F32)],
    )(*args)


def _act_lin(z1, vec, w2, with_stats):
    n4 = z1.shape[0]
    dout = w2.shape[1]

    def body(z_ref, v_ref, w_ref, *orefs):
        t = z_ref[...] * v_ref[0:1, :] + v_ref[1:2, :]
        t = jnp.where(t >= 0.0, t, v_ref[2:3, :] * t)
        z2 = jnp.dot(t, w_ref[...],
                     preferred_element_type=_F32) + v_ref[3:4, :dout]
        orefs[0][...] = z2
        if with_stats:
            @pl.when(pl.program_id(0) == 0)
            def _init():
                orefs[1][...] = jnp.zeros_like(orefs[1])
            orefs[1][...] += jnp.stack([jnp.sum(z2, 0), jnp.sum(z2 * z2, 0)])

    out_specs = [pl.BlockSpec((_BR4, dout), lambda i: (i, 0))]
    out_shape = [jax.ShapeDtypeStruct((n4, dout), _F32)]
    if with_stats:
        out_specs.append(pl.BlockSpec((2, dout), lambda i: (0, 0)))
        out_shape.append(jax.ShapeDtypeStruct((2, dout), _F32))
    res = pl.pallas_call(
        body,
        grid=(n4 // _BR4,),
        in_specs=[pl.BlockSpec((_BR4, 128), lambda i: (i, 0)),
                  pl.BlockSpec((4, 128), lambda i: (0, 0)),
                  pl.BlockSpec((128, dout), lambda i: (0, 0))],
        out_specs=out_specs,
        out_shape=out_shape,
    )(z1, vec, w2)
    return res if with_stats else res[0]


def _act_residual(z, vec, base):
    n4 = z.shape[0]

    def body(*refs):
        if base is None:
            z_ref, v_ref, o_ref = refs
        else:
            z_ref, v_ref, b_ref, o_ref = refs
        t = z_ref[...] * v_ref[0:1, :] + v_ref[1:2, :]
        t = jnp.where(t >= 0.0, t, v_ref[2:3, :] * t)
        if base is not None:
            t = b_ref[...] + t
        o_ref[...] = t

    in_specs = [pl.BlockSpec((_BR4, 128), lambda i: (i, 0)),
                pl.BlockSpec((3, 128), lambda i: (0, 0))]
    args = [z, vec]
    if base is not None:
        in_specs.append(pl.BlockSpec((_BR4, 128), lambda i: (i, 0)))
        args.append(base)
    return pl.pallas_call(
        body,
        grid=(n4 // _BR4,),
        in_specs=in_specs,
        out_specs=pl.BlockSpec((_BR4, 128), lambda i: (i, 0)),
        out_shape=jax.ShapeDtypeStruct((n4, 128), _F32),
    )(*args)


def _mlp2_block(x, w1, vec, w2):
    n, din = x.shape
    h = w1.shape[1]

    def bn_act(z, gm, bt, al):
        m = jnp.mean(z, 0, keepdims=True)
        v = jnp.mean(z * z, 0, keepdims=True) - m * m
        t = (z - m) * lax.rsqrt(v + _EPS) * gm + bt
        return jnp.where(t >= 0.0, t, al * t)

    def body(x_ref, w1_ref, v_ref, w2_ref, o_ref):
        z1 = jnp.dot(x_ref[...], w1_ref[...],
                     preferred_element_type=_F32) + v_ref[0:1, :]
        t = bn_act(z1, v_ref[1:2, :], v_ref[2:3, :], v_ref[3:4, :])
        z2 = jnp.dot(t, w2_ref[...],
                     preferred_element_type=_F32) + v_ref[4:5, :]
        o_ref[...] = bn_act(z2, v_ref[5:6, :], v_ref[6:7, :], v_ref[7:8, :])

    return pl.pallas_call(
        body,
        in_specs=[pl.BlockSpec((n, din), lambda: (0, 0)),
                  pl.BlockSpec((din, h), lambda: (0, 0)),
                  pl.BlockSpec((8, h), lambda: (0, 0)),
                  pl.BlockSpec((h, h), lambda: (0, 0))],
        out_specs=pl.BlockSpec((n, h), lambda: (0, 0)),
        out_shape=jax.ShapeDtypeStruct((n, h), _F32),
    )(x, w1, vec, w2)


def _node_update(node, accp, cntp, w1a, w1b, vec, w2, wproj, signs):
    n, h = node.shape
    pw = wproj.shape[1]

    def bn_act(z, gm, bt, al):
        m = jnp.mean(z, 0, keepdims=True)
        v = jnp.mean(z * z, 0, keepdims=True) - m * m
        t = (z - m) * lax.rsqrt(v + _EPS) * gm + bt
        return jnp.where(t >= 0.0, t, al * t)

    def body(nd_ref, ac_ref, ct_ref, wa_ref, wb_ref, v_ref, w2_ref,
             wp_ref, sg_ref, on_ref, op_ref):
        cnt = ct_ref[0] + ct_ref[1]
        inv = 1.0 / jnp.maximum(cnt[:, 0:1], 1.0)
        agg = (ac_ref[0] + ac_ref[1]) * inv
        nd = nd_ref[...]
        z1 = (jnp.dot(nd, wa_ref[...], preferred_element_type=_F32)
              + jnp.dot(agg, wb_ref[...], preferred_element_type=_F32)
              + v_ref[0:1, :])
        t = bn_act(z1, v_ref[1:2, :], v_ref[2:3, :], v_ref[3:4, :])
        z2 = jnp.dot(t, w2_ref[...],
                     preferred_element_type=_F32) + v_ref[4:5, :]
        nd_new = nd + bn_act(z2, v_ref[5:6, :], v_ref[6:7, :], v_ref[7:8, :])
        on_ref[...] = nd_new
        op_ref[...] = jnp.dot(nd_new, wp_ref[...],
                              preferred_element_type=_F32) * sg_ref[...]

    return pl.pallas_call(
        body,
        in_specs=[pl.BlockSpec((n, h), lambda: (0, 0)),
                  pl.BlockSpec((2, n, h), lambda: (0, 0, 0)),
                  pl.BlockSpec((2, n, 16), lambda: (0, 0, 0)),
                  pl.BlockSpec((h, h), lambda: (0, 0)),
                  pl.BlockSpec((h, h), lambda: (0, 0)),
                  pl.BlockSpec((8, h), lambda: (0, 0)),
                  pl.BlockSpec((h, h), lambda: (0, 0)),
                  pl.BlockSpec((h, pw), lambda: (0, 0)),
                  pl.BlockSpec((1, pw), lambda: (0, 0))],
        out_specs=[pl.BlockSpec((n, h), lambda: (0, 0)),
                   pl.BlockSpec((n, pw), lambda: (0, 0))],
        out_shape=[jax.ShapeDtypeStruct((n, h), _F32),
                   jax.ShapeDtypeStruct((n, pw), _F32)],
    )(node, accp, cntp, w1a, w1b, vec, w2, wproj, signs)


def _proj_call(node, wproj):
    n, h = node.shape
    pw = wproj.shape[1]

    def body(nd_ref, wp_ref, o_ref):
        o_ref[...] = jnp.dot(nd_ref[...], wp_ref[...],
                             preferred_element_type=_F32)

    return pl.pallas_call(
        body,
        in_specs=[pl.BlockSpec((n, h), lambda: (0, 0)),
                  pl.BlockSpec((h, pw), lambda: (0, 0))],
        out_specs=pl.BlockSpec((n, pw), lambda: (0, 0)),
        out_shape=jax.ShapeDtypeStruct((n, pw), _F32),
    )(node, wproj)


def _bd4(w):
    return block_diag(w, w, w, w)


def _t4(v):
    return jnp.tile(v, 4)


def _fold_bn(stats, n, bn, alpha):
    s = stats.reshape(2, 4, _H).sum(axis=1)
    m = s[0] / n
    v = s[1] / n - m * m
    a = bn["gamma"] * lax.rsqrt(v + _EPS)
    c = bn["beta"] - m * a
    return jnp.stack([a, c, jnp.full((_H,), alpha, _F32)])


def _vec4(ac3, b2, dout):
    row_b = jnp.zeros((128,), _F32).at[: 4 * dout].set(jnp.tile(b2, 4))
    return jnp.concatenate([jnp.tile(ac3, (1, 4)), row_b[None]], axis=0)


def _mlp_vec(p):
    return jnp.stack([
        p["lin1"]["b"], p["bn1"]["gamma"], p["bn1"]["beta"],
        jnp.full((_H,), p["pr1"]["alpha"], _F32),
        p["lin2"]["b"], p["bn2"]["gamma"], p["bn2"]["beta"],
        jnp.full((_H,), p["pr2"]["alpha"], _F32),
    ])


def kernel(x, edge_index, edge_attr, params):
    src = edge_index[0]
    dst = edge_index[1]
    dst2 = dst.reshape(_E // _CH, _CH)
    src2 = src.reshape(_E // _CH, _CH)
    layers = params["layers"]
    fin = params["final"]

    pe = params["node_emb"]
    node = _mlp2_block(x, pe["lin1"]["w"], _mlp_vec(pe), pe["lin2"]["w"])

    wm1 = layers[0]["msg"]["lin1"]["w"]
    proj = _proj_call(node, jnp.concatenate([wm1[:_H], wm1[_H:2 * _H]], 1))

    pg = params["edge_emb"]
    ea4 = edge_attr.reshape(_E4, 64)
    z1, s1 = _lin_stats(ea4, _bd4(pg["lin1"]["w"]),
                        _t4(pg["lin1"]["b"])[None])
    v1 = _fold_bn(s1, _E, pg["bn1"], pg["pr1"]["alpha"])
    z2, s2 = _act_lin(z1, _vec4(v1, pg["lin2"]["b"], _H),
                      _bd4(pg["lin2"]["w"]), True)
    v2 = _fold_bn(s2, _E, pg["bn2"], pg["pr2"]["alpha"])
    edge = _act_residual(z2, jnp.tile(v2, (1, 4)), None)

    cntp = _count_dst(dst2)

    for li, lp in enumerate(layers):
        mp, np_, ep = lp["msg"], lp["node"], lp["edge"]

        g1 = _gather_combine(proj[:, :_H], proj[:, _H:2 * _H], dst2, src2)
        wm = mp["lin1"]["w"]
        z1, s1 = _lin_stats(edge, _bd4(wm[2 * _H:]),
                            _t4(mp["lin1"]["b"])[None], g=g1)
        v1 = _fold_bn(s1, _E, mp["bn1"], mp["pr1"]["alpha"])
        z2, s2 = _act_lin(z1, _vec4(v1, mp["lin2"]["b"], _H),
                          _bd4(mp["lin2"]["w"]), True)
        ac = _fold_bn(s2, _E, mp["bn2"], mp["pr2"]["alpha"])
        accp = _scatter_msg(z2, dst2, ac)

        we = ep["lin1"]["w"][_H:]
        if li + 1 < len(layers):
            wn = layers[li + 1]["msg"]["lin1"]["w"]
            wp = jnp.concatenate([we, we, wn[:_H], wn[_H:2 * _H]], 1)
            signs = jnp.concatenate([
                jnp.full((1, _H), -1.0, _F32), jnp.full((1, _H), 1.0, _F32),
                jnp.full((1, 2 * _H), 1.0, _F32)], 1)
        else:
            wf = fin["lin1"]["w"][:_H]
            wp = jnp.concatenate([we, we, wf, wf], 1)
            signs = jnp.concatenate([
                jnp.full((1, _H), -1.0, _F32), jnp.full((1, _H), 1.0, _F32),
                jnp.full((1, _H), 1.0, _F32), jnp.full((1, _H), -1.0, _F32)],
                1)
        wn1 = np_["lin1"]["w"]
        node, proj4 = _node_update(node, accp, cntp, wn1[:_H], wn1[_H:],
                                   _mlp_vec(np_), np_["lin2"]["w"], wp, signs)

        d = _gather_combine(proj4[:, :_H], proj4[:, _H:2 * _H], dst2, src2)
        z1, s1 = _lin_stats(edge, _bd4(ep["lin1"]["w"][:_H]),
                            _t4(ep["lin1"]["b"])[None], g=d)
        v1 = _fold_bn(s1, _E, ep["bn1"], ep["pr1"]["alpha"])
        z2, s2 = _act_lin(z1, _vec4(v1, ep["lin2"]["b"], _H),
                          _bd4(ep["lin2"]["w"]), True)
        v2 = _fold_bn(s2, _E, ep["bn2"], ep["pr2"]["alpha"])
        edge = _act_residual(z2, jnp.tile(v2, (1, 4)), edge)
        proj = proj4[:, 2 * _H:]

    gf = _gather_combine(proj[:, :_H], proj[:, _H:], dst2, src2)
    z1, s1 = _lin_stats(edge, _bd4(fin["lin1"]["w"][_H:]),
                        _t4(fin["lin1"]["b"])[None], g=gf)
    v1 = _fold_bn(s1, _E, fin["bn1"], fin["pr1"]["alpha"])
    out4 = _act_lin(z1, _vec4(v1, fin["lin2"]["b"], 3),
                    _bd4(fin["lin2"]["w"]), False)
    return out4.reshape(_E, 3)


# speedup vs baseline: 3.9692x; 1.0232x over previous
import functools

import jax
import jax.numpy as jnp
from jax import lax
from jax.experimental import pallas as pl
from jax.experimental.pallas import tpu as pltpu
from jax.experimental.pallas import tpu_sc as plsc
from jax.scipy.linalg import block_diag

_N = 10000
_E = 320000
_H = 32
_EPS = 1e-5

_NC = 2
_NS = 16
_NW = _NC * _NS
_EPW = _E // _NW
_CH = 80
_CH4 = _CH // 4
_NCH = _EPW // _CH
_RPT = _N // _NS

_E4 = _E // 4
_BR4 = 4000
_F32 = jnp.float32


@functools.cache
def _sc_mesh():
    return plsc.VectorSubcoreMesh(core_axis_name="c", subcore_axis_name="s",
                                  num_cores=_NC, num_subcores=_NS)


@functools.cache
def _sc_gather_combine():
    @functools.partial(
        pl.kernel,
        out_type=jax.ShapeDtypeStruct((_E4, 128), _F32),
        mesh=_sc_mesh(),
        compiler_params=pltpu.CompilerParams(use_tc_tiling_on_sc=False),
        scratch_types=(
            [pltpu.VMEM((_NCH, _CH), jnp.int32)] * 2
            + [pltpu.VMEM((_CH, _H), _F32)] * 8
            + [pltpu.VMEM((_CH4, 128), _F32)] * 4
            + [pltpu.SemaphoreType.DMA] * 8
        ),
    )
    def gather_combine(a_hbm, b_hbm, dst2_hbm, src2_hbm, out_hbm,
                       idx_a, idx_b,
                       ra0, rb0, ra1, rb1, ra2, rb2, ra3, rb3,
                       sb0, sb1, sb2, sb3,
                       sem0, sem1, sem2, sem3,
                       semw0, semw1, semw2, semw3):
        wid = lax.axis_index("s") * _NC + lax.axis_index("c")
        wrow = wid * _NCH
        wrow4 = wid * (_EPW // 4)
        pltpu.sync_copy(dst2_hbm.at[pl.ds(wrow, _NCH)], idx_a)
        pltpu.sync_copy(src2_hbm.at[pl.ds(wrow, _NCH)], idx_b)

        bufs = ((ra0, rb0, sb0, sem0, semw0),
                (ra1, rb1, sb1, sem1, semw1),
                (ra2, rb2, sb2, sem2, semw2),
                (ra3, rb3, sb3, sem3, semw3))

        def fire(c, ra_, rb_, sem_):
            pltpu.async_copy(a_hbm.at[idx_a.at[c]], ra_, sem_)
            pltpu.async_copy(b_hbm.at[idx_b.at[c]], rb_, sem_)

        def wait_wb(c, sb_, semw_):
            pltpu.make_async_copy(
                sb_, out_hbm.at[pl.ds(wrow4 + c * _CH4, _CH4)], semw_).wait()

        def drain_process(c, ra_, rb_, sb_, sem_, semw_):
            pltpu.make_async_copy(a_hbm.at[idx_a.at[c]], ra_, sem_).wait()
            pltpu.make_async_copy(b_hbm.at[idx_b.at[c]], rb_, sem_).wait()

            @pl.when(c >= 4)
            def _():
                wait_wb(c - 4, sb_, semw_)

            def addq(q, c2):
                r0 = q * 4
                for k in range(4):
                    for h in range(2):
                        sl = pl.ds(h * 16, 16)
                        sb_[q, pl.ds(k * _H + h * 16, 16)] = (
                            ra_[r0 + k, sl] + rb_[r0 + k, sl])
                return c2

            lax.fori_loop(0, _CH4, addq, 0, unroll=2)
            pltpu.async_copy(sb_, out_hbm.at[pl.ds(wrow4 + c * _CH4, _CH4)],
                             semw_)

        for k in range(4):
            fire(k, bufs[k][0], bufs[k][1], bufs[k][3])

        def quad(p, carry):
            c0 = 4 * p
            for k in range(4):
                ra_, rb_, sb_, sem_, semw_ = bufs[k]
                c = c0 + k
                drain_process(c, ra_, rb_, sb_, sem_, semw_)

                @pl.when(c + 4 < _NCH)
                def _():
                    fire(c + 4, ra_, rb_, sem_)

            return carry

        lax.fori_loop(0, _NCH // 4, quad, 0)
        drain_process(_NCH - 1, *bufs[0])
        wait_wb(_NCH - 1, bufs[0][2], bufs[0][4])
        for k in range(1, 4):
            wait_wb(_NCH - 5 + k, bufs[k][2], bufs[k][4])

    return gather_combine


@functools.cache
def _sc_scatter_msg():
    @functools.partial(
        pl.kernel,
        out_type=jax.ShapeDtypeStruct((_NC, _N, _H), _F32),
        mesh=_sc_mesh(),
        compiler_params=pltpu.CompilerParams(use_tc_tiling_on_sc=False),
        scratch_types=(
            [pltpu.VMEM((_NCH, _CH), jnp.int32)]
            + [pltpu.VMEM((_CH4, 128), _F32)] * 4
            + [pltpu.VMEM((_CH, _H), _F32)] * 4
            + [pltpu.VMEM((_RPT, _H), _F32), pltpu.VMEM((3, _H), _F32),
               pltpu.VMEM_SHARED((_N, _H), _F32)]
            + [pltpu.SemaphoreType.DMA] * 8
        ),
    )
    def scatter_msg(z_hbm, dst2_hbm, ac_hbm, out_hbm,
                    idx_all, zp0, zp1, zp2, zp3, zs0, zs1, zs2, zs3,
                    tbuf, acv, acc,
                    semz0, semz1, semz2, semz3,
                    sems0, sems1, sems2, sems3):
        cid = lax.axis_index("c")
        sid = lax.axis_index("s")
        wid = sid * _NC + cid
        wrow = wid * _NCH
        wrow4 = wid * (_EPW // 4)

        def zrow(r, carry):
            for h in range(2):
                tbuf[r, pl.ds(h * 16, 16)] = jnp.zeros((16,), _F32)
            return carry

        lax.fori_loop(0, _RPT, zrow, 0)
        pltpu.sync_copy(tbuf, acc.at[pl.ds(sid * _RPT, _RPT)])
        pltpu.sync_copy(dst2_hbm.at[pl.ds(wrow, _NCH)], idx_all)
        plsc.subcore_barrier()

        pltpu.sync_copy(ac_hbm, acv)
        a_lo = acv[0, pl.ds(0, 16)]
        a_hi = acv[0, pl.ds(16, 16)]
        c_lo = acv[1, pl.ds(0, 16)]
        c_hi = acv[1, pl.ds(16, 16)]
        p_lo = acv[2, pl.ds(0, 16)]
        p_hi = acv[2, pl.ds(16, 16)]

        bufs = ((zp0, zs0, semz0, sems0), (zp1, zs1, semz1, sems1),
                (zp2, zs2, semz2, sems2), (zp3, zs3, semz3, sems3))

        def fire_z(c, zp_, semz_):
            pltpu.async_copy(z_hbm.at[pl.ds(wrow4 + c * _CH4, _CH4)], zp_,
                             semz_)

        def wait_scat(c, zs_, sems_):
            pltpu.make_async_copy(zs_, acc.at[idx_all.at[c]], sems_).wait()

        def process(c, zp_, zs_, semz_, sems_):
            pltpu.make_async_copy(
                z_hbm.at[pl.ds(wrow4 + c * _CH4, _CH4)], zp_, semz_).wait()

            @pl.when(c >= 4)
            def _():
                wait_scat(c - 4, zs_, sems_)

            def pq(q, c2):
                r0 = q * 4
                for k in range(4):
                    for h, (av, cv, pv) in enumerate(((a_lo, c_lo, p_lo),
                                                      (a_hi, c_hi, p_hi))):
                        v = zp_[q, pl.ds(k * _H + h * 16, 16)] * av + cv
                        zs_[r0 + k, pl.ds(h * 16, 16)] = jnp.where(
                            v >= 0.0, v, pv * v)
                return c2

            lax.fori_loop(0, _CH4, pq, 0, unroll=2)
            pltpu.async_copy(zs_, acc.at[idx_all.at[c]], sems_, add=True)

        for k in range(4):
            fire_z(k, bufs[k][0], bufs[k][2])

        def quad(p, carry):
            c0 = 4 * p
            for k in range(4):
                zp_, zs_, semz_, sems_ = bufs[k]
                c = c0 + k
                process(c, zp_, zs_, semz_, sems_)

                @pl.when(c + 4 < _NCH)
                def _():
                    fire_z(c + 4, zp_, semz_)

            return carry

        lax.fori_loop(0, _NCH // 4, quad, 0)
        process(_NCH - 1, *bufs[0])
        wait_scat(_NCH - 1, bufs[0][1], bufs[0][3])
        for k in range(1, 4):
            wait_scat(_NCH - 5 + k, bufs[k][1], bufs[k][3])
        plsc.subcore_barrier()

        pltpu.sync_copy(acc.at[pl.ds(sid * _RPT, _RPT)], tbuf)
        pltpu.sync_copy(tbuf, out_hbm.at[cid, pl.ds(sid * _RPT, _RPT)])

    return scatter_msg


@functools.cache
def _sc_count_dst():
    @functools.partial(
        pl.kernel,
        out_type=jax.ShapeDtypeStruct((_NC, _N, 16), _F32),
        mesh=_sc_mesh(),
        compiler_params=pltpu.CompilerParams(use_tc_tiling_on_sc=False),
        scratch_types=[
            pltpu.VMEM((_NCH, _CH), jnp.int32),
            pltpu.VMEM((_CH, 16), _F32),
            pltpu.VMEM((_RPT, 16), _F32),
            pltpu.VMEM_SHARED((_N, 16), _F32),
            pltpu.SemaphoreType.DMA,
        ],
    )
    def count_dst(dst2_hbm, out_hbm, idx_all, ones, tbuf, acc, sem):
        cid = lax.axis_index("c")
        sid = lax.axis_index("s")
        wid = sid * _NC + cid
        wrow = wid * _NCH

        def zrow(r, carry):
            tbuf[r, pl.ds(0, 16)] = jnp.zeros((16,), _F32)
            return carry

        lax.fori_loop(0, _RPT, zrow, 0)
        pltpu.sync_copy(tbuf, acc.at[pl.ds(sid * _RPT, _RPT)])
        pltpu.sync_copy(dst2_hbm.at[pl.ds(wrow, _NCH)], idx_all)

        lane = lax.broadcasted_iota(jnp.int32, (16,), 0)
        one_row = jnp.where(lane == 0, 1.0, 0.0).astype(_F32)

        def orow(r, carry):
            ones[r, pl.ds(0, 16)] = one_row
            return carry

        lax.fori_loop(0, _CH, orow, 0)
        plsc.subcore_barrier()

        def chunk(c, carry):
            pltpu.async_copy(ones, acc.at[idx_all.at[c]], sem, add=True)
            return carry

        lax.fori_loop(0, _NCH, chunk, 0)

        def drain(c, carry):
            pltpu.make_async_copy(ones, acc.at[idx_all.at[c]], sem).wait()
            return carry

        lax.fori_loop(0, _NCH, drain, 0)
        plsc.subcore_barrier()

        pltpu.sync_copy(acc.at[pl.ds(sid * _RPT, _RPT)], tbuf)
        pltpu.sync_copy(tbuf, out_hbm.at[cid, pl.ds(sid * _RPT, _RPT)])

    return count_dst


def _gather_combine(a, b, dst2, src2):
    return _sc_gather_combine()(a, b, dst2, src2)


def _scatter_msg(z, dst2, ac):
    return _sc_scatter_msg()(z, dst2, ac)


def _count_dst(dst2):
    return _sc_count_dst()(dst2)


def _lin_stats(e, w, b, g=None):
    n4, dp = e.shape

    def body(*refs):
        if g is None:
            e_ref, w_ref, b_ref, z_ref, s_ref = refs
            z = jnp.dot(e_ref[...], w_ref[...],
                        preferred_element_type=_F32) + b_ref[...]
        else:
            g_ref, e_ref, w_ref, b_ref, z_ref, s_ref = refs
            z = g_ref[...] + jnp.dot(e_ref[...], w_ref[...],
                                     preferred_element_type=_F32) + b_ref[...]
        z_ref[...] = z

        @pl.when(pl.program_id(0) == 0)
        def _init():
            s_ref[...] = jnp.zeros_like(s_ref)

        s_ref[...] += jnp.stack([jnp.sum(z, 0), jnp.sum(z * z, 0)])

    in_specs = [
        pl.BlockSpec((_BR4, dp), lambda i: (i, 0)),
        pl.BlockSpec((dp, 128), lambda i: (0, 0)),
        pl.BlockSpec((1, 128), lambda i: (0, 0)),
    ]
    args = [e, w, b]
    if g is not None:
        in_specs.insert(0, pl.BlockSpec((_BR4, 128), lambda i: (i, 0)))
        args.insert(0, g)
    return pl.pallas_call(
        body,
        grid=(n4 // _BR4,),
        in_specs=in_specs,
        out_specs=[pl.BlockSpec((_BR4, 128), lambda i: (i, 0)),
                   pl.BlockSpec((2, 128), lambda i: (0, 0))],
        out_shape=[jax.ShapeDtypeStruct((n4, 128), _F32),
                   jax.ShapeDtypeStruct((2, 128), _F32)],
    )(*args)


def _act_lin(z1, vec, w2, with_stats):
    n4 = z1.shape[0]
    dout = w2.shape[1]

    def body(z_ref, v_ref, w_ref, *orefs):
        t = z_ref[...] * v_ref[0:1, :] + v_ref[1:2, :]
        t = jnp.where(t >= 0.0, t, v_ref[2:3, :] * t)
        z2 = jnp.dot(t, w_ref[...],
                     preferred_element_type=_F32) + v_ref[3:4, :dout]
        orefs[0][...] = z2
        if with_stats:
            @pl.when(pl.program_id(0) == 0)
            def _init():
                orefs[1][...] = jnp.zeros_like(orefs[1])
            orefs[1][...] += jnp.stack([jnp.sum(z2, 0), jnp.sum(z2 * z2, 0)])

    out_specs = [pl.BlockSpec((_BR4, dout), lambda i: (i, 0))]
    out_shape = [jax.ShapeDtypeStruct((n4, dout), _F32)]
    if with_stats:
        out_specs.append(pl.BlockSpec((2, dout), lambda i: (0, 0)))
        out_shape.append(jax.ShapeDtypeStruct((2, dout), _F32))
    res = pl.pallas_call(
        body,
        grid=(n4 // _BR4,),
        in_specs=[pl.BlockSpec((_BR4, 128), lambda i: (i, 0)),
                  pl.BlockSpec((4, 128), lambda i: (0, 0)),
                  pl.BlockSpec((128, dout), lambda i: (0, 0))],
        out_specs=out_specs,
        out_shape=out_shape,
    )(z1, vec, w2)
    return res if with_stats else res[0]


def _act_residual(z, vec, base):
    n4 = z.shape[0]

    def body(*refs):
        if base is None:
            z_ref, v_ref, o_ref = refs
        else:
            z_ref, v_ref, b_ref, o_ref = refs
        t = z_ref[...] * v_ref[0:1, :] + v_ref[1:2, :]
        t = jnp.where(t >= 0.0, t, v_ref[2:3, :] * t)
        if base is not None:
            t = b_ref[...] + t
        o_ref[...] = t

    in_specs = [pl.BlockSpec((_BR4, 128), lambda i: (i, 0)),
                pl.BlockSpec((3, 128), lambda i: (0, 0))]
    args = [z, vec]
    if base is not None:
        in_specs.append(pl.BlockSpec((_BR4, 128), lambda i: (i, 0)))
        args.append(base)
    return pl.pallas_call(
        body,
        grid=(n4 // _BR4,),
        in_specs=in_specs,
        out_specs=pl.BlockSpec((_BR4, 128), lambda i: (i, 0)),
        out_shape=jax.ShapeDtypeStruct((n4, 128), _F32),
    )(*args)


def _mlp2_block(x, w1, vec, w2):
    n, din = x.shape
    h = w1.shape[1]

    def bn_act(z, gm, bt, al):
        m = jnp.mean(z, 0, keepdims=True)
        v = jnp.mean(z * z, 0, keepdims=True) - m * m
        t = (z - m) * lax.rsqrt(v + _EPS) * gm + bt
        return jnp.where(t >= 0.0, t, al * t)

    def body(x_ref, w1_ref, v_ref, w2_ref, o_ref):
        z1 = jnp.dot(x_ref[...], w1_ref[...],
                     preferred_element_type=_F32) + v_ref[0:1, :]
        t = bn_act(z1, v_ref[1:2, :], v_ref[2:3, :], v_ref[3:4, :])
        z2 = jnp.dot(t, w2_ref[...],
                     preferred_element_type=_F32) + v_ref[4:5, :]
        o_ref[...] = bn_act(z2, v_ref[5:6, :], v_ref[6:7, :], v_ref[7:8, :])

    return pl.pallas_call(
        body,
        in_specs=[pl.BlockSpec((n, din), lambda: (0, 0)),
                  pl.BlockSpec((din, h), lambda: (0, 0)),
                  pl.BlockSpec((8, h), lambda: (0, 0)),
                  pl.BlockSpec((h, h), lambda: (0, 0))],
        out_specs=pl.BlockSpec((n, h), lambda: (0, 0)),
        out_shape=jax.ShapeDtypeStruct((n, h), _F32),
    )(x, w1, vec, w2)


def _node_update(node, accp, cntp, w1a, w1b, vec, w2, wproj, signs):
    n, h = node.shape
    pw = wproj.shape[1]

    def bn_act(z, gm, bt, al):
        m = jnp.mean(z, 0, keepdims=True)
        v = jnp.mean(z * z, 0, keepdims=True) - m * m
        t = (z - m) * lax.rsqrt(v + _EPS) * gm + bt
        return jnp.where(t >= 0.0, t, al * t)

    def body(nd_ref, ac_ref, ct_ref, wa_ref, wb_ref, v_ref, w2_ref,
             wp_ref, sg_ref, on_ref, op_ref):
        cnt = ct_ref[0] + ct_ref[1]
        inv = 1.0 / jnp.maximum(cnt[:, 0:1], 1.0)
        agg = (ac_ref[0] + ac_ref[1]) * inv
        nd = nd_ref[...]
        z1 = (jnp.dot(nd, wa_ref[...], preferred_element_type=_F32)
              + jnp.dot(agg, wb_ref[...], preferred_element_type=_F32)
              + v_ref[0:1, :])
        t = bn_act(z1, v_ref[1:2, :], v_ref[2:3, :], v_ref[3:4, :])
        z2 = jnp.dot(t, w2_ref[...],
                     preferred_element_type=_F32) + v_ref[4:5, :]
        nd_new = nd + bn_act(z2, v_ref[5:6, :], v_ref[6:7, :], v_ref[7:8, :])
        on_ref[...] = nd_new
        op_ref[...] = jnp.dot(nd_new, wp_ref[...],
                              preferred_element_type=_F32) * sg_ref[...]

    return pl.pallas_call(
        body,
        in_specs=[pl.BlockSpec((n, h), lambda: (0, 0)),
                  pl.BlockSpec((2, n, h), lambda: (0, 0, 0)),
                  pl.BlockSpec((2, n, 16), lambda: (0, 0, 0)),
                  pl.BlockSpec((h, h), lambda: (0, 0)),
                  pl.BlockSpec((h, h), lambda: (0, 0)),
                  pl.BlockSpec((8, h), lambda: (0, 0)),
                  pl.BlockSpec((h, h), lambda: (0, 0)),
                  pl.BlockSpec((h, pw), lambda: (0, 0)),
                  pl.BlockSpec((1, pw), lambda: (0, 0))],
        out_specs=[pl.BlockSpec((n, h), lambda: (0, 0)),
                   pl.BlockSpec((n, pw), lambda: (0, 0))],
        out_shape=[jax.ShapeDtypeStruct((n, h), _F32),
                   jax.ShapeDtypeStruct((n, pw), _F32)],
    )(node, accp, cntp, w1a, w1b, vec, w2, wproj, signs)


def _proj_call(node, wproj):
    n, h = node.shape
    pw = wproj.shape[1]

    def body(nd_ref, wp_ref, o_ref):
        o_ref[...] = jnp.dot(nd_ref[...], wp_ref[...],
                             preferred_element_type=_F32)

    return pl.pallas_call(
        body,
        in_specs=[pl.BlockSpec((n, h), lambda: (0, 0)),
                  pl.BlockSpec((h, pw), lambda: (0, 0))],
        out_specs=pl.BlockSpec((n, pw), lambda: (0, 0)),
        out_shape=jax.ShapeDtypeStruct((n, pw), _F32),
    )(node, wproj)


def _bd4(w):
    return block_diag(w, w, w, w)


def _t4(v):
    return jnp.tile(v, 4)


def _fold_bn(stats, n, bn, alpha):
    s = stats.reshape(2, 4, _H).sum(axis=1)
    m = s[0] / n
    v = s[1] / n - m * m
    a = bn["gamma"] * lax.rsqrt(v + _EPS)
    c = bn["beta"] - m * a
    return jnp.stack([a, c, jnp.full((_H,), alpha, _F32)])


def _vec4(ac3, b2, dout):
    row_b = jnp.zeros((128,), _F32).at[: 4 * dout].set(jnp.tile(b2, 4))
    return jnp.concatenate([jnp.tile(ac3, (1, 4)), row_b[None]], axis=0)


def _mlp_vec(p):
    return jnp.stack([
        p["lin1"]["b"], p["bn1"]["gamma"], p["bn1"]["beta"],
        jnp.full((_H,), p["pr1"]["alpha"], _F32),
        p["lin2"]["b"], p["bn2"]["gamma"], p["bn2"]["beta"],
        jnp.full((_H,), p["pr2"]["alpha"], _F32),
    ])


def kernel(x, edge_index, edge_attr, params):
    src = edge_index[0]
    dst = edge_index[1]
    dst2 = dst.reshape(_E // _CH, _CH)
    src2 = src.reshape(_E // _CH, _CH)
    layers = params["layers"]
    fin = params["final"]

    pe = params["node_emb"]
    node = _mlp2_block(x, pe["lin1"]["w"], _mlp_vec(pe), pe["lin2"]["w"])

    wm1 = layers[0]["msg"]["lin1"]["w"]
    proj = _proj_call(node, jnp.concatenate([wm1[:_H], wm1[_H:2 * _H]], 1))

    pg = params["edge_emb"]
    ea4 = edge_attr.reshape(_E4, 64)
    z1, s1 = _lin_stats(ea4, _bd4(pg["lin1"]["w"]),
                        _t4(pg["lin1"]["b"])[None])
    v1 = _fold_bn(s1, _E, pg["bn1"], pg["pr1"]["alpha"])
    z2, s2 = _act_lin(z1, _vec4(v1, pg["lin2"]["b"], _H),
                      _bd4(pg["lin2"]["w"]), True)
    v2 = _fold_bn(s2, _E, pg["bn2"], pg["pr2"]["alpha"])
    edge = _act_residual(z2, jnp.tile(v2, (1, 4)), None)

    cntp = _count_dst(dst2)

    for li, lp in enumerate(layers):
        mp, np_, ep = lp["msg"], lp["node"], lp["edge"]

        g1 = _gather_combine(proj[:, :_H], proj[:, _H:2 * _H], dst2, src2)
        wm = mp["lin1"]["w"]
        z1, s1 = _lin_stats(edge, _bd4(wm[2 * _H:]),
                            _t4(mp["lin1"]["b"])[None], g=g1)
        v1 = _fold_bn(s1, _E, mp["bn1"], mp["pr1"]["alpha"])
        z2, s2 = _act_lin(z1, _vec4(v1, mp["lin2"]["b"], _H),
                          _bd4(mp["lin2"]["w"]), True)
        ac = _fold_bn(s2, _E, mp["bn2"], mp["pr2"]["alpha"])
        accp = _scatter_msg(z2, dst2, ac)

        we = ep["lin1"]["w"][_H:]
        if li + 1 < len(layers):
            wn = layers[li + 1]["msg"]["lin1"]["w"]
            wp = jnp.concatenate([we, we, wn[:_H], wn[_H:2 * _H]], 1)
            signs = jnp.concatenate([
                jnp.full((1, _H), -1.0, _F32), jnp.full((1, _H), 1.0, _F32),
                jnp.full((1, 2 * _H), 1.0, _F32)], 1)
        else:
            wf = fin["lin1"]["w"][:_H]
            wp = jnp.concatenate([we, we, wf, wf], 1)
            signs = jnp.concatenate([
                jnp.full((1, _H), -1.0, _F32), jnp.full((1, _H), 1.0, _F32),
                jnp.full((1, _H), 1.0, _F32), jnp.full((1, _H), -1.0, _F32)],
                1)
        wn1 = np_["lin1"]["w"]
        node, proj4 = _node_update(node, accp, cntp, wn1[:_H], wn1[_H:],
                                   _mlp_vec(np_), np_["lin2"]["w"], wp, signs)

        d = _gather_combine(proj4[:, :_H], proj4[:, _H:2 * _H], dst2, src2)
        z1, s1 = _lin_stats(edge, _bd4(ep["lin1"]["w"][:_H]),
                            _t4(ep["lin1"]["b"])[None], g=d)
        v1 = _fold_bn(s1, _E, ep["bn1"], ep["pr1"]["alpha"])
        z2, s2 = _act_lin(z1, _vec4(v1, ep["lin2"]["b"], _H),
                          _bd4(ep["lin2"]["w"]), True)
        v2 = _fold_bn(s2, _E, ep["bn2"], ep["pr2"]["alpha"])
        edge = _act_residual(z2, jnp.tile(v2, (1, 4)), edge)
        proj = proj4[:, 2 * _H:]

    gf = _gather_combine(proj[:, :_H], proj[:, _H:], dst2, src2)
    z1, s1 = _lin_stats(edge, _bd4(fin["lin1"]["w"][_H:]),
                        _t4(fin["lin1"]["b"])[None], g=gf)
    v1 = _fold_bn(s1, _E, fin["bn1"], fin["pr1"]["alpha"])
    out4 = _act_lin(z1, _vec4(v1, fin["lin2"]["b"], 3),
                    _bd4(fin["lin2"]["w"]), False)
    return out4.reshape(_E, 3)
